# baseline (speedup 1.0000x reference)
import jax
import jax.numpy as jnp
from jax.experimental import pallas as pl

N = 10000
HEADS = 8
DH = 16
L = 4


def _mm_kernel(x_ref, w_ref, o_ref):
    o_ref[...] = jnp.dot(x_ref[...], w_ref[...], preferred_element_type=jnp.float32)


def _mm(x, w):
    return pl.pallas_call(
        _mm_kernel,
        out_shape=jax.ShapeDtypeStruct((x.shape[0], w.shape[1]), jnp.float32),
    )(x, w)


def kernel(h, edge_index, node_graph_id, snorm_n, W_embed, Ws, a_srcs, a_dsts, W_ro, W_pred, b_pred):
    src = edge_index[0]
    dst = edge_index[1]
    x = _mm(h.astype(jnp.float32), W_embed)
    for i in range(L):
        x_in = x
        W, a_s, a_d = Ws[i], a_srcs[i], a_dsts[i]
        Wh = (x @ W).reshape(N, HEADS, DH)
        el = jnp.sum(Wh * a_s[None, :, :], axis=-1)
        er = jnp.sum(Wh * a_d[None, :, :], axis=-1)
        e = jax.nn.leaky_relu(el[src] + er[dst], 0.2)
        emax = jax.ops.segment_max(e, dst, num_segments=N)
        emax = jnp.where(jnp.isfinite(emax), emax, 0.0)
        ex = jnp.exp(e - emax[dst])
        denom = jax.ops.segment_sum(ex, dst, num_segments=N) + 1e-9
        alpha = ex / denom[dst]
        msg = alpha[:, :, None] * Wh[src]
        out = jax.ops.segment_sum(msg, dst, num_segments=N).reshape(N, HEADS * DH)
        out = out * snorm_n
        out = jax.nn.elu(out)
        x = out + x_in
    hn = x @ W_ro
    G = 64
    sums = jax.ops.segment_sum(hn, node_graph_id, num_segments=G)
    counts = jax.ops.segment_sum(jnp.ones((N, 1), dtype=jnp.float32), node_graph_id, num_segments=G)
    hg = sums / jnp.maximum(counts, 1.0)
    return hg @ W_pred + b_pred


# speedup vs baseline: 67.8274x; 67.8274x over previous
import functools

import jax
import jax.numpy as jnp
from jax import lax
from jax.experimental import pallas as pl
from jax.experimental.pallas import tpu as pltpu
from jax.experimental.pallas import tpu_sc as plsc

N = 10000
E = 320000
H_DIM = 128
HEADS = 8
DH = 16
L = 4
G = 64

NC = 2
NS = 16
NW = NC * NS
EPW = E // NW
RPS = 624
TAIL = N - NS * RPS

BN = 2000
GRID = N // BN

CK_E = 1000
NIT_E = EPW // CK_E
CK_M = 200
NIT_M = EPW // CK_M

_f32 = jnp.float32


def _x0_body(h_ref, we_ref, x_ref):
    x_ref[...] = jnp.dot(h_ref[...], we_ref[...], preferred_element_type=_f32)


def _x0_call(h, W_embed):
    return pl.pallas_call(
        _x0_body,
        grid=(GRID,),
        in_specs=[_ROW_SPEC, _W_SPEC],
        out_specs=_ROW_SPEC,
        out_shape=jax.ShapeDtypeStruct((N, H_DIM), _f32),
    )(h, W_embed)


def _al_body(xp_ref, o0_ref, o1_ref, sn_ref, w_ref, aser_ref,
             x_ref, wh_ref, el_ref, er_ref, m_ref):
    i = pl.program_id(0)
    z = (o0_ref[...] + o1_ref[...]) * sn_ref[...]
    z = jnp.where(z > 0, z, jnp.exp(z) - 1.0)
    x = xp_ref[...] + z
    x_ref[...] = x
    wh = jnp.dot(x, w_ref[...], preferred_element_type=_f32)
    wh_ref[...] = wh
    eler = jnp.dot(wh, aser_ref[...], preferred_element_type=_f32)
    el_ref[...] = eler[:, :16]
    er_ref[...] = eler[:, 16:]
    blkmax = jnp.max(eler[:, :16], axis=0, keepdims=True)

    @pl.when(i == 0)
    def _():
        m_ref[...] = blkmax

    @pl.when(i > 0)
    def _():
        m_ref[...] = jnp.maximum(m_ref[...], blkmax)


_A_OUT_SHAPES = [
    jax.ShapeDtypeStruct((N, H_DIM), _f32),
    jax.ShapeDtypeStruct((N, H_DIM), _f32),
    jax.ShapeDtypeStruct((N, 16), _f32),
    jax.ShapeDtypeStruct((N, 16), _f32),
    jax.ShapeDtypeStruct((1, 16), _f32),
]

_A_OUT_SPECS = [
    pl.BlockSpec((BN, H_DIM), lambda i: (i, 0)),
    pl.BlockSpec((BN, H_DIM), lambda i: (i, 0)),
    pl.BlockSpec((BN, 16), lambda i: (i, 0)),
    pl.BlockSpec((BN, 16), lambda i: (i, 0)),
    pl.BlockSpec((1, 16), lambda i: (0, 0)),
]

_W_SPEC = pl.BlockSpec((H_DIM, H_DIM), lambda i: (0, 0))
_ASER_SPEC = pl.BlockSpec((H_DIM, 32), lambda i: (0, 0))
_ROW_SPEC = pl.BlockSpec((BN, H_DIM), lambda i: (i, 0))
_COL_SPEC = pl.BlockSpec((BN, 1), lambda i: (i, 0))


def _al_call(x_prev, o0, o1, snorm, W, aser):
    return pl.pallas_call(
        _al_body,
        grid=(GRID,),
        in_specs=[_ROW_SPEC, _ROW_SPEC, _ROW_SPEC, _COL_SPEC, _W_SPEC, _ASER_SPEC],
        out_specs=_A_OUT_SPECS,
        out_shape=_A_OUT_SHAPES,
    )(x_prev, o0, o1, snorm, W, aser)


def _ro_body(xp_ref, o0_ref, o1_ref, sn_ref, gid_ref, wro_ref, wp_ref, bp_ref,
             out_ref, sums_ref, cnts_ref):
    i = pl.program_id(0)
    z = (o0_ref[...] + o1_ref[...]) * sn_ref[...]
    z = jnp.where(z > 0, z, jnp.exp(z) - 1.0)
    y = xp_ref[...] + z
    hn = jnp.dot(y, wro_ref[...], preferred_element_type=_f32)
    gid = gid_ref[...]
    iota = lax.broadcasted_iota(jnp.int32, (BN, G), 1)
    p = (gid == iota).astype(_f32)
    psum = lax.dot_general(p, hn, (((0,), (0,)), ((), ())),
                           preferred_element_type=_f32)
    ones = jnp.ones((BN, H_DIM), _f32)
    pcnt = lax.dot_general(p, ones, (((0,), (0,)), ((), ())),
                           preferred_element_type=_f32)

    @pl.when(i == 0)
    def _():
        sums_ref[...] = psum
        cnts_ref[...] = pcnt

    @pl.when(i > 0)
    def _():
        sums_ref[...] = sums_ref[...] + psum
        cnts_ref[...] = cnts_ref[...] + pcnt

    @pl.when(i == GRID - 1)
    def _():
        hg = sums_ref[...] / jnp.maximum(cnts_ref[...], 1.0)
        out_ref[...] = jnp.dot(hg, wp_ref[...], preferred_element_type=_f32) + bp_ref[...]


def _ro_call(x_prev, o0, o1, snorm, gid2d, W_ro, W_pred, b_pred2d):
    return pl.pallas_call(
        _ro_body,
        grid=(GRID,),
        in_specs=[
            _ROW_SPEC, _ROW_SPEC, _ROW_SPEC, _COL_SPEC,
            pl.BlockSpec((BN, 1), lambda i: (i, 0)),
            pl.BlockSpec((H_DIM, H_DIM), lambda i: (0, 0)),
            pl.BlockSpec((H_DIM, 1), lambda i: (0, 0)),
            pl.BlockSpec((1, 1), lambda i: (0, 0)),
        ],
        out_specs=pl.BlockSpec((G, 1), lambda i: (0, 0)),
        out_shape=jax.ShapeDtypeStruct((G, 1), _f32),
        scratch_shapes=[
            pltpu.VMEM((G, H_DIM), _f32),
            pltpu.VMEM((G, H_DIM), _f32),
        ],
    )(x_prev, o0, o1, snorm, gid2d, W_ro, W_pred, b_pred2d)


def _r_body(d0_ref, d1_ref, r_ref):
    r_ref[...] = 1.0 / (d0_ref[...] + d1_ref[...] + 1e-9)


def _r_call(den0, den1):
    return pl.pallas_call(
        _r_body,
        grid=(GRID,),
        in_specs=[pl.BlockSpec((BN, 16), lambda i: (i, 0)),
                  pl.BlockSpec((BN, 16), lambda i: (i, 0))],
        out_specs=pl.BlockSpec((BN, 16), lambda i: (i, 0)),
        out_shape=jax.ShapeDtypeStruct((N, 16), _f32),
    )(den0, den1)


_MESH = plsc.VectorSubcoreMesh(core_axis_name="c", subcore_axis_name="s")


@functools.partial(
    pl.kernel,
    out_type=[
        jax.ShapeDtypeStruct((E, 16), _f32),
        jax.ShapeDtypeStruct((N, 16), _f32),
        jax.ShapeDtypeStruct((N, 16), _f32),
    ],
    mesh=_MESH,
    compiler_params=pltpu.CompilerParams(use_tc_tiling_on_sc=False),
    scratch_types=[
        pltpu.VMEM((CK_E,), jnp.int32),
        pltpu.VMEM((CK_E,), jnp.int32),
        pltpu.VMEM((CK_E, 16), _f32),
        pltpu.VMEM((CK_E, 16), _f32),
        pltpu.VMEM((CK_E, 16), _f32),
        pltpu.VMEM((16,), _f32),
        pltpu.VMEM_SHARED((N, 16), _f32),
        pltpu.SemaphoreType.DMA,
        pltpu.SemaphoreType.DMA,
    ],
)
def _e_kernel(src_hbm, dst_hbm, el_hbm, er_hbm, m_hbm, z16_hbm,
              w_hbm, den0_hbm, den1_hbm,
              idx_s, idx_d, abuf, bbuf, wbuf, mvec, den_sh,
              sem1, sem2):
    c = lax.axis_index("c")
    s = lax.axis_index("s")
    wid = s * NC + c

    pltpu.sync_copy(z16_hbm.at[pl.ds(s * RPS, RPS)], den_sh.at[pl.ds(s * RPS, RPS)])

    @pl.when(s == NS - 1)
    def _():
        pltpu.sync_copy(z16_hbm.at[pl.ds(NS * RPS, TAIL)],
                        den_sh.at[pl.ds(NS * RPS, TAIL)])

    pltpu.sync_copy(m_hbm, mvec)
    plsc.subcore_barrier()

    mv = mvec[...]

    def chunk(j, carry):
        base = wid * EPW + j * CK_E
        pltpu.sync_copy(src_hbm.at[pl.ds(base, CK_E)], idx_s)
        pltpu.sync_copy(dst_hbm.at[pl.ds(base, CK_E)], idx_d)
        cp1 = pltpu.async_copy(el_hbm.at[idx_s], abuf, sem1)
        cp2 = pltpu.async_copy(er_hbm.at[idx_d], bbuf, sem2)
        cp1.wait()
        cp2.wait()

        def vbody(i, carry2):
            a = abuf[i, :]
            b = bbuf[i, :]
            e = a + b
            e = jnp.maximum(e, 0.2 * e)
            cb = mv + b
            cb = jnp.maximum(cb, 0.2 * cb)
            wbuf[i, :] = jnp.exp(e - cb)
            return carry2

        lax.fori_loop(0, CK_E, vbody, 0, unroll=4)
        pltpu.sync_copy(wbuf, w_hbm.at[pl.ds(base, CK_E)])
        pltpu.sync_copy(wbuf, den_sh.at[idx_d], add=True)
        return carry

    lax.fori_loop(0, NIT_E, chunk, 0)

    plsc.subcore_barrier()

    @pl.when(c == 0)
    def _():
        pltpu.sync_copy(den_sh.at[pl.ds(s * RPS, RPS)], den0_hbm.at[pl.ds(s * RPS, RPS)])

        @pl.when(s == NS - 1)
        def _():
            pltpu.sync_copy(den_sh.at[pl.ds(NS * RPS, TAIL)],
                            den0_hbm.at[pl.ds(NS * RPS, TAIL)])

    @pl.when(c == 1)
    def _():
        pltpu.sync_copy(den_sh.at[pl.ds(s * RPS, RPS)], den1_hbm.at[pl.ds(s * RPS, RPS)])

        @pl.when(s == NS - 1)
        def _():
            pltpu.sync_copy(den_sh.at[pl.ds(NS * RPS, TAIL)],
                            den1_hbm.at[pl.ds(NS * RPS, TAIL)])


@functools.partial(
    pl.kernel,
    out_type=[
        jax.ShapeDtypeStruct((N, H_DIM), _f32),
        jax.ShapeDtypeStruct((N, H_DIM), _f32),
    ],
    mesh=_MESH,
    compiler_params=pltpu.CompilerParams(use_tc_tiling_on_sc=False),
    scratch_types=[
        pltpu.VMEM((CK_M,), jnp.int32),
        pltpu.VMEM((CK_M,), jnp.int32),
        pltpu.VMEM((CK_M, H_DIM), _f32),
        pltpu.VMEM((CK_M, 16), _f32),
        pltpu.VMEM((CK_M, 16), _f32),
        pltpu.VMEM_SHARED((N, H_DIM), _f32),
        pltpu.SemaphoreType.DMA,
        pltpu.SemaphoreType.DMA,
    ],
)
def _m_kernel(src_hbm, dst_hbm, wh_hbm, w_hbm, r_hbm, z128_hbm,
              o0_hbm, o1_hbm,
              idx_s, idx_d, gbuf, wb, rg, out_sh,
              sem1, sem2):
    c = lax.axis_index("c")
    s = lax.axis_index("s")
    wid = s * NC + c

    pltpu.sync_copy(z128_hbm.at[pl.ds(s * RPS, RPS)], out_sh.at[pl.ds(s * RPS, RPS)])

    @pl.when(s == NS - 1)
    def _():
        pltpu.sync_copy(z128_hbm.at[pl.ds(NS * RPS, TAIL)],
                        out_sh.at[pl.ds(NS * RPS, TAIL)])

    plsc.subcore_barrier()

    def chunk(j, carry):
        base = wid * EPW + j * CK_M
        pltpu.sync_copy(src_hbm.at[pl.ds(base, CK_M)], idx_s)
        pltpu.sync_copy(dst_hbm.at[pl.ds(base, CK_M)], idx_d)
        cp1 = pltpu.async_copy(wh_hbm.at[idx_s], gbuf, sem1)
        cp2 = pltpu.async_copy(r_hbm.at[idx_d], rg, sem2)
        pltpu.sync_copy(w_hbm.at[pl.ds(base, CK_M)], wb)
        cp1.wait()
        cp2.wait()

        def vbody(i, carry2):
            aw = wb[i, :] * rg[i, :]
            for hd in range(HEADS):
                gbuf[i, 16 * hd:16 * (hd + 1)] = (
                    gbuf[i, 16 * hd:16 * (hd + 1)] * aw[hd])
            return carry2

        lax.fori_loop(0, CK_M, vbody, 0)
        pltpu.sync_copy(gbuf, out_sh.at[idx_d], add=True)
        return carry

    lax.fori_loop(0, NIT_M, chunk, 0)

    plsc.subcore_barrier()

    @pl.when(c == 0)
    def _():
        pltpu.sync_copy(out_sh.at[pl.ds(s * RPS, RPS)], o0_hbm.at[pl.ds(s * RPS, RPS)])

        @pl.when(s == NS - 1)
        def _():
            pltpu.sync_copy(out_sh.at[pl.ds(NS * RPS, TAIL)],
                            o0_hbm.at[pl.ds(NS * RPS, TAIL)])

    @pl.when(c == 1)
    def _():
        pltpu.sync_copy(out_sh.at[pl.ds(s * RPS, RPS)], o1_hbm.at[pl.ds(s * RPS, RPS)])

        @pl.when(s == NS - 1)
        def _():
            pltpu.sync_copy(out_sh.at[pl.ds(NS * RPS, TAIL)],
                            o1_hbm.at[pl.ds(NS * RPS, TAIL)])


def kernel(h, edge_index, node_graph_id, snorm_n, W_embed, Ws, a_srcs, a_dsts,
           W_ro, W_pred, b_pred):
    src = edge_index[0]
    dst = edge_index[1]

    head_of = jnp.arange(H_DIM, dtype=jnp.int32) // DH
    delta = (head_of[:, None] == jnp.arange(HEADS)[None, :]).astype(_f32)
    asv = a_srcs.reshape(L, H_DIM)[:, :, None] * delta[None]
    adv = a_dsts.reshape(L, H_DIM)[:, :, None] * delta[None]
    aser = jnp.concatenate([asv, asv, adv, adv], axis=2)

    zeros16 = jnp.zeros((N, 16), _f32)
    zeros128 = jnp.zeros((N, H_DIM), _f32)
    snorm = snorm_n.astype(_f32)

    x0 = _x0_call(h.astype(_f32), W_embed)

    def layer(l, carry):
        x, o0, o1 = carry
        W_l = lax.dynamic_index_in_dim(Ws, l, 0, keepdims=False)
        aser_l = lax.dynamic_index_in_dim(aser, l, 0, keepdims=False)
        x, wh, el, er, m = _al_call(x, o0, o1, snorm, W_l, aser_l)
        w, den0, den1 = _e_kernel(src, dst, el, er, m.reshape(16), zeros16)
        r = _r_call(den0, den1)
        o0, o1 = _m_kernel(src, dst, wh, w, r, zeros128)
        return (x, o0, o1)

    x, o0, o1 = lax.fori_loop(0, L, layer, (x0, zeros128, zeros128))

    gid2d = node_graph_id.reshape(N, 1)
    b2d = b_pred.reshape(1, 1)
    return _ro_call(x, o0, o1, snorm, gid2d, W_ro, W_pred, b2d)


# speedup vs baseline: 68.1261x; 1.0044x over previous
import functools

import jax
import jax.numpy as jnp
from jax import lax
from jax.experimental import pallas as pl
from jax.experimental.pallas import tpu as pltpu
from jax.experimental.pallas import tpu_sc as plsc

N = 10000
E = 320000
H_DIM = 128
HEADS = 8
DH = 16
L = 4
G = 64

NC = 2
NS = 16
NW = NC * NS
EPW = E // NW
RPS = 624
TAIL = N - NS * RPS

BN = 2000
GRID = N // BN

CK_E = 2000
NIT_E = EPW // CK_E
CK_M = 200
NIT_M = EPW // CK_M

_f32 = jnp.float32


def _x0_body(h_ref, we_ref, x_ref):
    x_ref[...] = jnp.dot(h_ref[...], we_ref[...], preferred_element_type=_f32)


def _x0_call(h, W_embed):
    return pl.pallas_call(
        _x0_body,
        grid=(GRID,),
        in_specs=[_ROW_SPEC, _W_SPEC],
        out_specs=_ROW_SPEC,
        out_shape=jax.ShapeDtypeStruct((N, H_DIM), _f32),
    )(h, W_embed)


def _al_body(xp_ref, o0_ref, o1_ref, sn_ref, w_ref, aser_ref,
             x_ref, wh_ref, el_ref, er_ref, m_ref):
    i = pl.program_id(0)
    z = (o0_ref[...] + o1_ref[...]) * sn_ref[...]
    z = jnp.where(z > 0, z, jnp.exp(z) - 1.0)
    x = xp_ref[...] + z
    x_ref[...] = x
    wh = jnp.dot(x, w_ref[...], preferred_element_type=_f32)
    wh_ref[...] = wh
    eler = jnp.dot(wh, aser_ref[...], preferred_element_type=_f32)
    el_ref[...] = eler[:, :16]
    er_ref[...] = eler[:, 16:]
    blkmax = jnp.max(eler[:, :16], axis=0, keepdims=True)

    @pl.when(i == 0)
    def _():
        m_ref[...] = blkmax

    @pl.when(i > 0)
    def _():
        m_ref[...] = jnp.maximum(m_ref[...], blkmax)


_A_OUT_SHAPES = [
    jax.ShapeDtypeStruct((N, H_DIM), _f32),
    jax.ShapeDtypeStruct((N, H_DIM), _f32),
    jax.ShapeDtypeStruct((N, 16), _f32),
    jax.ShapeDtypeStruct((N, 16), _f32),
    jax.ShapeDtypeStruct((1, 16), _f32),
]

_A_OUT_SPECS = [
    pl.BlockSpec((BN, H_DIM), lambda i: (i, 0)),
    pl.BlockSpec((BN, H_DIM), lambda i: (i, 0)),
    pl.BlockSpec((BN, 16), lambda i: (i, 0)),
    pl.BlockSpec((BN, 16), lambda i: (i, 0)),
    pl.BlockSpec((1, 16), lambda i: (0, 0)),
]

_W_SPEC = pl.BlockSpec((H_DIM, H_DIM), lambda i: (0, 0))
_ASER_SPEC = pl.BlockSpec((H_DIM, 32), lambda i: (0, 0))
_ROW_SPEC = pl.BlockSpec((BN, H_DIM), lambda i: (i, 0))
_COL_SPEC = pl.BlockSpec((BN, 1), lambda i: (i, 0))


def _al_call(x_prev, o0, o1, snorm, W, aser):
    return pl.pallas_call(
        _al_body,
        grid=(GRID,),
        in_specs=[_ROW_SPEC, _ROW_SPEC, _ROW_SPEC, _COL_SPEC, _W_SPEC, _ASER_SPEC],
        out_specs=_A_OUT_SPECS,
        out_shape=_A_OUT_SHAPES,
    )(x_prev, o0, o1, snorm, W, aser)


def _ro_body(xp_ref, o0_ref, o1_ref, sn_ref, gid_ref, wro_ref, wp_ref, bp_ref,
             out_ref, sums_ref, cnts_ref):
    i = pl.program_id(0)
    z = (o0_ref[...] + o1_ref[...]) * sn_ref[...]
    z = jnp.where(z > 0, z, jnp.exp(z) - 1.0)
    y = xp_ref[...] + z
    hn = jnp.dot(y, wro_ref[...], preferred_element_type=_f32)
    gid = gid_ref[...]
    iota = lax.broadcasted_iota(jnp.int32, (BN, G), 1)
    p = (gid == iota).astype(_f32)
    psum = lax.dot_general(p, hn, (((0,), (0,)), ((), ())),
                           preferred_element_type=_f32)
    ones = jnp.ones((BN, H_DIM), _f32)
    pcnt = lax.dot_general(p, ones, (((0,), (0,)), ((), ())),
                           preferred_element_type=_f32)

    @pl.when(i == 0)
    def _():
        sums_ref[...] = psum
        cnts_ref[...] = pcnt

    @pl.when(i > 0)
    def _():
        sums_ref[...] = sums_ref[...] + psum
        cnts_ref[...] = cnts_ref[...] + pcnt

    @pl.when(i == GRID - 1)
    def _():
        hg = sums_ref[...] / jnp.maximum(cnts_ref[...], 1.0)
        out_ref[...] = jnp.dot(hg, wp_ref[...], preferred_element_type=_f32) + bp_ref[...]


def _ro_call(x_prev, o0, o1, snorm, gid2d, W_ro, W_pred, b_pred2d):
    return pl.pallas_call(
        _ro_body,
        grid=(GRID,),
        in_specs=[
            _ROW_SPEC, _ROW_SPEC, _ROW_SPEC, _COL_SPEC,
            pl.BlockSpec((BN, 1), lambda i: (i, 0)),
            pl.BlockSpec((H_DIM, H_DIM), lambda i: (0, 0)),
            pl.BlockSpec((H_DIM, 1), lambda i: (0, 0)),
            pl.BlockSpec((1, 1), lambda i: (0, 0)),
        ],
        out_specs=pl.BlockSpec((G, 1), lambda i: (0, 0)),
        out_shape=jax.ShapeDtypeStruct((G, 1), _f32),
        scratch_shapes=[
            pltpu.VMEM((G, H_DIM), _f32),
            pltpu.VMEM((G, H_DIM), _f32),
        ],
    )(x_prev, o0, o1, snorm, gid2d, W_ro, W_pred, b_pred2d)


def _r_body(d0_ref, d1_ref, r_ref):
    r_ref[...] = 1.0 / (d0_ref[...] + d1_ref[...] + 1e-9)


def _r_call(den0, den1):
    return pl.pallas_call(
        _r_body,
        grid=(GRID,),
        in_specs=[pl.BlockSpec((BN, 16), lambda i: (i, 0)),
                  pl.BlockSpec((BN, 16), lambda i: (i, 0))],
        out_specs=pl.BlockSpec((BN, 16), lambda i: (i, 0)),
        out_shape=jax.ShapeDtypeStruct((N, 16), _f32),
    )(den0, den1)


_MESH = plsc.VectorSubcoreMesh(core_axis_name="c", subcore_axis_name="s")


@functools.partial(
    pl.kernel,
    out_type=[
        jax.ShapeDtypeStruct((E, 16), _f32),
        jax.ShapeDtypeStruct((N, 16), _f32),
        jax.ShapeDtypeStruct((N, 16), _f32),
    ],
    mesh=_MESH,
    compiler_params=pltpu.CompilerParams(use_tc_tiling_on_sc=False),
    scratch_types=[
        pltpu.VMEM((CK_E,), jnp.int32),
        pltpu.VMEM((CK_E,), jnp.int32),
        pltpu.VMEM((CK_E, 16), _f32),
        pltpu.VMEM((CK_E, 16), _f32),
        pltpu.VMEM((CK_E, 16), _f32),
        pltpu.VMEM((16,), _f32),
        pltpu.VMEM_SHARED((N, 16), _f32),
        pltpu.SemaphoreType.DMA,
        pltpu.SemaphoreType.DMA,
    ],
)
def _e_kernel(src_hbm, dst_hbm, el_hbm, er_hbm, m_hbm, z16_hbm,
              w_hbm, den0_hbm, den1_hbm,
              idx_s, idx_d, abuf, bbuf, wbuf, mvec, den_sh,
              sem1, sem2):
    c = lax.axis_index("c")
    s = lax.axis_index("s")
    wid = s * NC + c

    pltpu.sync_copy(z16_hbm.at[pl.ds(s * RPS, RPS)], den_sh.at[pl.ds(s * RPS, RPS)])

    @pl.when(s == NS - 1)
    def _():
        pltpu.sync_copy(z16_hbm.at[pl.ds(NS * RPS, TAIL)],
                        den_sh.at[pl.ds(NS * RPS, TAIL)])

    pltpu.sync_copy(m_hbm, mvec)
    plsc.subcore_barrier()

    mv = mvec[...]

    def chunk(j, carry):
        base = wid * EPW + j * CK_E
        pltpu.sync_copy(src_hbm.at[pl.ds(base, CK_E)], idx_s)
        pltpu.sync_copy(dst_hbm.at[pl.ds(base, CK_E)], idx_d)
        cp1 = pltpu.async_copy(el_hbm.at[idx_s], abuf, sem1)
        cp2 = pltpu.async_copy(er_hbm.at[idx_d], bbuf, sem2)
        cp1.wait()
        cp2.wait()

        def vbody(i, carry2):
            a = abuf[i, :]
            b = bbuf[i, :]
            e = a + b
            e = jnp.maximum(e, 0.2 * e)
            cb = mv + b
            cb = jnp.maximum(cb, 0.2 * cb)
            wbuf[i, :] = jnp.exp(e - cb)
            return carry2

        lax.fori_loop(0, CK_E, vbody, 0, unroll=4)
        pltpu.sync_copy(wbuf, w_hbm.at[pl.ds(base, CK_E)])
        pltpu.sync_copy(wbuf, den_sh.at[idx_d], add=True)
        return carry

    lax.fori_loop(0, NIT_E, chunk, 0)

    plsc.subcore_barrier()

    @pl.when(c == 0)
    def _():
        pltpu.sync_copy(den_sh.at[pl.ds(s * RPS, RPS)], den0_hbm.at[pl.ds(s * RPS, RPS)])

        @pl.when(s == NS - 1)
        def _():
            pltpu.sync_copy(den_sh.at[pl.ds(NS * RPS, TAIL)],
                            den0_hbm.at[pl.ds(NS * RPS, TAIL)])

    @pl.when(c == 1)
    def _():
        pltpu.sync_copy(den_sh.at[pl.ds(s * RPS, RPS)], den1_hbm.at[pl.ds(s * RPS, RPS)])

        @pl.when(s == NS - 1)
        def _():
            pltpu.sync_copy(den_sh.at[pl.ds(NS * RPS, TAIL)],
                            den1_hbm.at[pl.ds(NS * RPS, TAIL)])


@functools.partial(
    pl.kernel,
    out_type=[
        jax.ShapeDtypeStruct((N, H_DIM), _f32),
        jax.ShapeDtypeStruct((N, H_DIM), _f32),
    ],
    mesh=_MESH,
    compiler_params=pltpu.CompilerParams(use_tc_tiling_on_sc=False),
    scratch_types=[
        pltpu.VMEM((CK_M,), jnp.int32),
        pltpu.VMEM((CK_M,), jnp.int32),
        pltpu.VMEM((CK_M, H_DIM), _f32),
        pltpu.VMEM((CK_M, 16), _f32),
        pltpu.VMEM((CK_M, 16), _f32),
        pltpu.VMEM_SHARED((N, H_DIM), _f32),
        pltpu.SemaphoreType.DMA,
        pltpu.SemaphoreType.DMA,
    ],
)
def _m_kernel(src_hbm, dst_hbm, wh_hbm, w_hbm, r_hbm, z128_hbm,
              o0_hbm, o1_hbm,
              idx_s, idx_d, gbuf, wb, rg, out_sh,
              sem1, sem2):
    c = lax.axis_index("c")
    s = lax.axis_index("s")
    wid = s * NC + c

    pltpu.sync_copy(z128_hbm.at[pl.ds(s * RPS, RPS)], out_sh.at[pl.ds(s * RPS, RPS)])

    @pl.when(s == NS - 1)
    def _():
        pltpu.sync_copy(z128_hbm.at[pl.ds(NS * RPS, TAIL)],
                        out_sh.at[pl.ds(NS * RPS, TAIL)])

    plsc.subcore_barrier()

    def chunk(j, carry):
        base = wid * EPW + j * CK_M
        pltpu.sync_copy(src_hbm.at[pl.ds(base, CK_M)], idx_s)
        pltpu.sync_copy(dst_hbm.at[pl.ds(base, CK_M)], idx_d)
        cp1 = pltpu.async_copy(wh_hbm.at[idx_s], gbuf, sem1)
        cp2 = pltpu.async_copy(r_hbm.at[idx_d], rg, sem2)
        pltpu.sync_copy(w_hbm.at[pl.ds(base, CK_M)], wb)
        cp1.wait()
        cp2.wait()

        def vbody(i, carry2):
            aw = wb[i, :] * rg[i, :]
            for hd in range(HEADS):
                gbuf[i, 16 * hd:16 * (hd + 1)] = (
                    gbuf[i, 16 * hd:16 * (hd + 1)] * aw[hd])
            return carry2

        lax.fori_loop(0, CK_M, vbody, 0, unroll=2)
        pltpu.sync_copy(gbuf, out_sh.at[idx_d], add=True)
        return carry

    lax.fori_loop(0, NIT_M, chunk, 0)

    plsc.subcore_barrier()

    @pl.when(c == 0)
    def _():
        pltpu.sync_copy(out_sh.at[pl.ds(s * RPS, RPS)], o0_hbm.at[pl.ds(s * RPS, RPS)])

        @pl.when(s == NS - 1)
        def _():
            pltpu.sync_copy(out_sh.at[pl.ds(NS * RPS, TAIL)],
                            o0_hbm.at[pl.ds(NS * RPS, TAIL)])

    @pl.when(c == 1)
    def _():
        pltpu.sync_copy(out_sh.at[pl.ds(s * RPS, RPS)], o1_hbm.at[pl.ds(s * RPS, RPS)])

        @pl.when(s == NS - 1)
        def _():
            pltpu.sync_copy(out_sh.at[pl.ds(NS * RPS, TAIL)],
                            o1_hbm.at[pl.ds(NS * RPS, TAIL)])


def kernel(h, edge_index, node_graph_id, snorm_n, W_embed, Ws, a_srcs, a_dsts,
           W_ro, W_pred, b_pred):
    src = edge_index[0]
    dst = edge_index[1]

    head_of = jnp.arange(H_DIM, dtype=jnp.int32) // DH
    delta = (head_of[:, None] == jnp.arange(HEADS)[None, :]).astype(_f32)
    asv = a_srcs.reshape(L, H_DIM)[:, :, None] * delta[None]
    adv = a_dsts.reshape(L, H_DIM)[:, :, None] * delta[None]
    aser = jnp.concatenate([asv, asv, adv, adv], axis=2)

    zeros16 = jnp.zeros((N, 16), _f32)
    zeros128 = jnp.zeros((N, H_DIM), _f32)
    snorm = snorm_n.astype(_f32)

    x0 = _x0_call(h.astype(_f32), W_embed)

    def layer(l, carry):
        x, o0, o1 = carry
        W_l = lax.dynamic_index_in_dim(Ws, l, 0, keepdims=False)
        aser_l = lax.dynamic_index_in_dim(aser, l, 0, keepdims=False)
        x, wh, el, er, m = _al_call(x, o0, o1, snorm, W_l, aser_l)
        w, den0, den1 = _e_kernel(src, dst, el, er, m.reshape(16), zeros16)
        r = _r_call(den0, den1)
        o0, o1 = _m_kernel(src, dst, wh, w, r, zeros128)
        return (x, o0, o1)

    x, o0, o1 = lax.fori_loop(0, L, layer, (x0, zeros128, zeros128))

    gid2d = node_graph_id.reshape(N, 1)
    b2d = b_pred.reshape(1, 1)
    return _ro_call(x, o0, o1, snorm, gid2d, W_ro, W_pred, b2d)


# speedup vs baseline: 99.8924x; 1.4663x over previous
import functools

import jax
import jax.numpy as jnp
from jax import lax
from jax.experimental import pallas as pl
from jax.experimental.pallas import tpu as pltpu
from jax.experimental.pallas import tpu_sc as plsc

N = 10000
E = 320000
H_DIM = 128
HEADS = 8
DH = 16
L = 4
G = 64

NC = 2
NS = 16
NW = NC * NS
EPW = E // NW
RPS = 624
TAIL = N - NS * RPS

BN = 2000
GRID = N // BN

CK_E = 2000
NIT_E = EPW // CK_E
CK_M = 200
NIT_M = EPW // CK_M

_f32 = jnp.float32


def _x0_body(h_ref, we_ref, x_ref):
    x_ref[...] = jnp.dot(h_ref[...], we_ref[...], preferred_element_type=_f32)


def _x0_call(h, W_embed):
    return pl.pallas_call(
        _x0_body,
        grid=(GRID,),
        in_specs=[_ROW_SPEC, _W_SPEC],
        out_specs=_ROW_SPEC,
        out_shape=jax.ShapeDtypeStruct((N, H_DIM), _f32),
    )(h, W_embed)


def _al_body(xp_ref, o0_ref, o1_ref, sn_ref, w_ref, aser_ref,
             x_ref, wh_ref, el_ref, er_ref, m_ref):
    i = pl.program_id(0)
    z = (o0_ref[...] + o1_ref[...]) * sn_ref[...]
    z = jnp.where(z > 0, z, jnp.exp(z) - 1.0)
    x = xp_ref[...] + z
    x_ref[...] = x
    wh = jnp.dot(x, w_ref[...], preferred_element_type=_f32)
    wh_ref[...] = wh
    eler = jnp.dot(wh, aser_ref[...], preferred_element_type=_f32)
    el_ref[...] = eler[:, :16]
    er_ref[...] = eler[:, 16:]
    blkmax = jnp.max(eler[:, :16], axis=0, keepdims=True)

    @pl.when(i == 0)
    def _():
        m_ref[...] = blkmax

    @pl.when(i > 0)
    def _():
        m_ref[...] = jnp.maximum(m_ref[...], blkmax)


_A_OUT_SHAPES = [
    jax.ShapeDtypeStruct((N, H_DIM), _f32),
    jax.ShapeDtypeStruct((N, H_DIM), _f32),
    jax.ShapeDtypeStruct((N, 16), _f32),
    jax.ShapeDtypeStruct((N, 16), _f32),
    jax.ShapeDtypeStruct((1, 16), _f32),
]

_A_OUT_SPECS = [
    pl.BlockSpec((BN, H_DIM), lambda i: (i, 0)),
    pl.BlockSpec((BN, H_DIM), lambda i: (i, 0)),
    pl.BlockSpec((BN, 16), lambda i: (i, 0)),
    pl.BlockSpec((BN, 16), lambda i: (i, 0)),
    pl.BlockSpec((1, 16), lambda i: (0, 0)),
]

_W_SPEC = pl.BlockSpec((H_DIM, H_DIM), lambda i: (0, 0))
_ASER_SPEC = pl.BlockSpec((H_DIM, 32), lambda i: (0, 0))
_ROW_SPEC = pl.BlockSpec((BN, H_DIM), lambda i: (i, 0))
_COL_SPEC = pl.BlockSpec((BN, 1), lambda i: (i, 0))


def _al_call(x_prev, o0, o1, snorm, W, aser):
    return pl.pallas_call(
        _al_body,
        grid=(GRID,),
        in_specs=[_ROW_SPEC, _ROW_SPEC, _ROW_SPEC, _COL_SPEC, _W_SPEC, _ASER_SPEC],
        out_specs=_A_OUT_SPECS,
        out_shape=_A_OUT_SHAPES,
    )(x_prev, o0, o1, snorm, W, aser)


def _ro_body(xp_ref, o0_ref, o1_ref, sn_ref, gid_ref, wro_ref, wp_ref, bp_ref,
             out_ref, sums_ref, cnts_ref):
    i = pl.program_id(0)
    z = (o0_ref[...] + o1_ref[...]) * sn_ref[...]
    z = jnp.where(z > 0, z, jnp.exp(z) - 1.0)
    y = xp_ref[...] + z
    hn = jnp.dot(y, wro_ref[...], preferred_element_type=_f32)
    gid = gid_ref[...]
    iota = lax.broadcasted_iota(jnp.int32, (BN, G), 1)
    p = (gid == iota).astype(_f32)
    psum = lax.dot_general(p, hn, (((0,), (0,)), ((), ())),
                           preferred_element_type=_f32)
    ones = jnp.ones((BN, H_DIM), _f32)
    pcnt = lax.dot_general(p, ones, (((0,), (0,)), ((), ())),
                           preferred_element_type=_f32)

    @pl.when(i == 0)
    def _():
        sums_ref[...] = psum
        cnts_ref[...] = pcnt

    @pl.when(i > 0)
    def _():
        sums_ref[...] = sums_ref[...] + psum
        cnts_ref[...] = cnts_ref[...] + pcnt

    @pl.when(i == GRID - 1)
    def _():
        hg = sums_ref[...] / jnp.maximum(cnts_ref[...], 1.0)
        out_ref[...] = jnp.dot(hg, wp_ref[...], preferred_element_type=_f32) + bp_ref[...]


def _ro_call(x_prev, o0, o1, snorm, gid2d, W_ro, W_pred, b_pred2d):
    return pl.pallas_call(
        _ro_body,
        grid=(GRID,),
        in_specs=[
            _ROW_SPEC, _ROW_SPEC, _ROW_SPEC, _COL_SPEC,
            pl.BlockSpec((BN, 1), lambda i: (i, 0)),
            pl.BlockSpec((H_DIM, H_DIM), lambda i: (0, 0)),
            pl.BlockSpec((H_DIM, 1), lambda i: (0, 0)),
            pl.BlockSpec((1, 1), lambda i: (0, 0)),
        ],
        out_specs=pl.BlockSpec((G, 1), lambda i: (0, 0)),
        out_shape=jax.ShapeDtypeStruct((G, 1), _f32),
        scratch_shapes=[
            pltpu.VMEM((G, H_DIM), _f32),
            pltpu.VMEM((G, H_DIM), _f32),
        ],
    )(x_prev, o0, o1, snorm, gid2d, W_ro, W_pred, b_pred2d)


def _r_body(d0_ref, d1_ref, r_ref):
    r_ref[...] = 1.0 / (d0_ref[...] + d1_ref[...] + 1e-9)


def _r_call(den0, den1):
    return pl.pallas_call(
        _r_body,
        grid=(GRID,),
        in_specs=[pl.BlockSpec((BN, 16), lambda i: (i, 0)),
                  pl.BlockSpec((BN, 16), lambda i: (i, 0))],
        out_specs=pl.BlockSpec((BN, 16), lambda i: (i, 0)),
        out_shape=jax.ShapeDtypeStruct((N, 16), _f32),
    )(den0, den1)


_MESH = plsc.VectorSubcoreMesh(core_axis_name="c", subcore_axis_name="s")


@functools.partial(
    pl.kernel,
    out_type=[
        jax.ShapeDtypeStruct((E, 16), _f32),
        jax.ShapeDtypeStruct((N, 16), _f32),
        jax.ShapeDtypeStruct((N, 16), _f32),
    ],
    mesh=_MESH,
    compiler_params=pltpu.CompilerParams(use_tc_tiling_on_sc=False),
    scratch_types=[
        pltpu.VMEM((CK_E,), jnp.int32),
        pltpu.VMEM((CK_E,), jnp.int32),
        pltpu.VMEM((CK_E, 16), _f32),
        pltpu.VMEM((CK_E, 16), _f32),
        pltpu.VMEM((CK_E, 16), _f32),
        pltpu.VMEM((16,), _f32),
        pltpu.VMEM_SHARED((N, 16), _f32),
        pltpu.SemaphoreType.DMA,
        pltpu.SemaphoreType.DMA,
    ],
)
def _e_kernel(src_hbm, dst_hbm, el_hbm, er_hbm, m_hbm, z16_hbm,
              w_hbm, den0_hbm, den1_hbm,
              idx_s, idx_d, abuf, bbuf, wbuf, mvec, den_sh,
              sem1, sem2):
    c = lax.axis_index("c")
    s = lax.axis_index("s")
    wid = s * NC + c

    pltpu.sync_copy(z16_hbm.at[pl.ds(s * RPS, RPS)], den_sh.at[pl.ds(s * RPS, RPS)])

    @pl.when(s == NS - 1)
    def _():
        pltpu.sync_copy(z16_hbm.at[pl.ds(NS * RPS, TAIL)],
                        den_sh.at[pl.ds(NS * RPS, TAIL)])

    pltpu.sync_copy(m_hbm, mvec)
    plsc.subcore_barrier()

    mv = mvec[...]

    def chunk(j, carry):
        base = wid * EPW + j * CK_E
        pltpu.sync_copy(src_hbm.at[pl.ds(base, CK_E)], idx_s)
        pltpu.sync_copy(dst_hbm.at[pl.ds(base, CK_E)], idx_d)
        cp1 = pltpu.async_copy(el_hbm.at[idx_s], abuf, sem1)
        cp2 = pltpu.async_copy(er_hbm.at[idx_d], bbuf, sem2)
        cp1.wait()
        cp2.wait()

        @plsc.parallel_loop(0, CK_E, unroll=8)
        def vbody(i):
            a = abuf[i, :]
            b = bbuf[i, :]
            e = a + b
            e = jnp.maximum(e, 0.2 * e)
            cb = mv + b
            cb = jnp.maximum(cb, 0.2 * cb)
            wbuf[i, :] = jnp.exp(e - cb)
        pltpu.sync_copy(wbuf, w_hbm.at[pl.ds(base, CK_E)])
        pltpu.sync_copy(wbuf, den_sh.at[idx_d], add=True)
        return carry

    lax.fori_loop(0, NIT_E, chunk, 0)

    plsc.subcore_barrier()

    @pl.when(c == 0)
    def _():
        pltpu.sync_copy(den_sh.at[pl.ds(s * RPS, RPS)], den0_hbm.at[pl.ds(s * RPS, RPS)])

        @pl.when(s == NS - 1)
        def _():
            pltpu.sync_copy(den_sh.at[pl.ds(NS * RPS, TAIL)],
                            den0_hbm.at[pl.ds(NS * RPS, TAIL)])

    @pl.when(c == 1)
    def _():
        pltpu.sync_copy(den_sh.at[pl.ds(s * RPS, RPS)], den1_hbm.at[pl.ds(s * RPS, RPS)])

        @pl.when(s == NS - 1)
        def _():
            pltpu.sync_copy(den_sh.at[pl.ds(NS * RPS, TAIL)],
                            den1_hbm.at[pl.ds(NS * RPS, TAIL)])


@functools.partial(
    pl.kernel,
    out_type=[
        jax.ShapeDtypeStruct((N, H_DIM), _f32),
        jax.ShapeDtypeStruct((N, H_DIM), _f32),
    ],
    mesh=_MESH,
    compiler_params=pltpu.CompilerParams(use_tc_tiling_on_sc=False),
    scratch_types=[
        pltpu.VMEM((CK_M,), jnp.int32),
        pltpu.VMEM((CK_M,), jnp.int32),
        pltpu.VMEM((CK_M, H_DIM), _f32),
        pltpu.VMEM((CK_M, 16), _f32),
        pltpu.VMEM((CK_M, 16), _f32),
        pltpu.VMEM_SHARED((N, H_DIM), _f32),
        pltpu.SemaphoreType.DMA,
        pltpu.SemaphoreType.DMA,
    ],
)
def _m_kernel(src_hbm, dst_hbm, wh_hbm, w_hbm, r_hbm, z128_hbm,
              o0_hbm, o1_hbm,
              idx_s, idx_d, gbuf, wb, rg, out_sh,
              sem1, sem2):
    c = lax.axis_index("c")
    s = lax.axis_index("s")
    wid = s * NC + c

    pltpu.sync_copy(z128_hbm.at[pl.ds(s * RPS, RPS)], out_sh.at[pl.ds(s * RPS, RPS)])

    @pl.when(s == NS - 1)
    def _():
        pltpu.sync_copy(z128_hbm.at[pl.ds(NS * RPS, TAIL)],
                        out_sh.at[pl.ds(NS * RPS, TAIL)])

    plsc.subcore_barrier()

    def chunk(j, carry):
        base = wid * EPW + j * CK_M
        pltpu.sync_copy(src_hbm.at[pl.ds(base, CK_M)], idx_s)
        pltpu.sync_copy(dst_hbm.at[pl.ds(base, CK_M)], idx_d)
        cp1 = pltpu.async_copy(wh_hbm.at[idx_s], gbuf, sem1)
        cp2 = pltpu.async_copy(r_hbm.at[idx_d], rg, sem2)
        pltpu.sync_copy(w_hbm.at[pl.ds(base, CK_M)], wb)
        cp1.wait()
        cp2.wait()

        @plsc.parallel_loop(0, CK_M, unroll=4)
        def vbody(i):
            aw = wb[i, :] * rg[i, :]
            for hd in range(HEADS):
                gbuf[i, 16 * hd:16 * (hd + 1)] = (
                    gbuf[i, 16 * hd:16 * (hd + 1)] * aw[hd])
        pltpu.sync_copy(gbuf, out_sh.at[idx_d], add=True)
        return carry

    lax.fori_loop(0, NIT_M, chunk, 0)

    plsc.subcore_barrier()

    @pl.when(c == 0)
    def _():
        pltpu.sync_copy(out_sh.at[pl.ds(s * RPS, RPS)], o0_hbm.at[pl.ds(s * RPS, RPS)])

        @pl.when(s == NS - 1)
        def _():
            pltpu.sync_copy(out_sh.at[pl.ds(NS * RPS, TAIL)],
                            o0_hbm.at[pl.ds(NS * RPS, TAIL)])

    @pl.when(c == 1)
    def _():
        pltpu.sync_copy(out_sh.at[pl.ds(s * RPS, RPS)], o1_hbm.at[pl.ds(s * RPS, RPS)])

        @pl.when(s == NS - 1)
        def _():
            pltpu.sync_copy(out_sh.at[pl.ds(NS * RPS, TAIL)],
                            o1_hbm.at[pl.ds(NS * RPS, TAIL)])


def kernel(h, edge_index, node_graph_id, snorm_n, W_embed, Ws, a_srcs, a_dsts,
           W_ro, W_pred, b_pred):
    src = edge_index[0]
    dst = edge_index[1]

    head_of = jnp.arange(H_DIM, dtype=jnp.int32) // DH
    delta = (head_of[:, None] == jnp.arange(HEADS)[None, :]).astype(_f32)
    asv = a_srcs.reshape(L, H_DIM)[:, :, None] * delta[None]
    adv = a_dsts.reshape(L, H_DIM)[:, :, None] * delta[None]
    aser = jnp.concatenate([asv, asv, adv, adv], axis=2)

    zeros16 = jnp.zeros((N, 16), _f32)
    zeros128 = jnp.zeros((N, H_DIM), _f32)
    snorm = snorm_n.astype(_f32)

    x0 = _x0_call(h.astype(_f32), W_embed)

    def layer(l, carry):
        x, o0, o1 = carry
        W_l = lax.dynamic_index_in_dim(Ws, l, 0, keepdims=False)
        aser_l = lax.dynamic_index_in_dim(aser, l, 0, keepdims=False)
        x, wh, el, er, m = _al_call(x, o0, o1, snorm, W_l, aser_l)
        w, den0, den1 = _e_kernel(src, dst, el, er, m.reshape(16), zeros16)
        r = _r_call(den0, den1)
        o0, o1 = _m_kernel(src, dst, wh, w, r, zeros128)
        return (x, o0, o1)

    x, o0, o1 = lax.fori_loop(0, L, layer, (x0, zeros128, zeros128))

    gid2d = node_graph_id.reshape(N, 1)
    b2d = b_pred.reshape(1, 1)
    return _ro_call(x, o0, o1, snorm, gid2d, W_ro, W_pred, b2d)


# speedup vs baseline: 111.4414x; 1.1156x over previous
import functools

import jax
import jax.numpy as jnp
from jax import lax
from jax.experimental import pallas as pl
from jax.experimental.pallas import tpu as pltpu
from jax.experimental.pallas import tpu_sc as plsc

N = 10000
E = 320000
H_DIM = 128
HEADS = 8
DH = 16
L = 4
G = 64

NC = 2
NS = 16
NW = NC * NS
EPW = E // NW
RPS = 624
TAIL = N - NS * RPS

BN = 2000
GRID = N // BN

CK_E = 2000
NIT_E = EPW // CK_E
CK_M = 80
NIT_M = EPW // CK_M

_f32 = jnp.float32


def _x0_body(h_ref, we_ref, x_ref):
    x_ref[...] = jnp.dot(h_ref[...], we_ref[...], preferred_element_type=_f32)


def _x0_call(h, W_embed):
    return pl.pallas_call(
        _x0_body,
        grid=(GRID,),
        in_specs=[_ROW_SPEC, _W_SPEC],
        out_specs=_ROW_SPEC,
        out_shape=jax.ShapeDtypeStruct((N, H_DIM), _f32),
    )(h, W_embed)


def _al_body(xp_ref, o0_ref, o1_ref, sn_ref, w_ref, aser_ref,
             x_ref, wh_ref, el_ref, er_ref, m_ref):
    i = pl.program_id(0)
    z = (o0_ref[...] + o1_ref[...]) * sn_ref[...]
    z = jnp.where(z > 0, z, jnp.exp(z) - 1.0)
    x = xp_ref[...] + z
    x_ref[...] = x
    wh = jnp.dot(x, w_ref[...], preferred_element_type=_f32)
    wh_ref[...] = wh
    eler = jnp.dot(wh, aser_ref[...], preferred_element_type=_f32)
    el_ref[...] = eler[:, :16]
    er_ref[...] = eler[:, 16:]
    blkmax = jnp.max(eler[:, :16], axis=0, keepdims=True)

    @pl.when(i == 0)
    def _():
        m_ref[...] = blkmax

    @pl.when(i > 0)
    def _():
        m_ref[...] = jnp.maximum(m_ref[...], blkmax)


_A_OUT_SHAPES = [
    jax.ShapeDtypeStruct((N, H_DIM), _f32),
    jax.ShapeDtypeStruct((N, H_DIM), _f32),
    jax.ShapeDtypeStruct((N, 16), _f32),
    jax.ShapeDtypeStruct((N, 16), _f32),
    jax.ShapeDtypeStruct((1, 16), _f32),
]

_A_OUT_SPECS = [
    pl.BlockSpec((BN, H_DIM), lambda i: (i, 0)),
    pl.BlockSpec((BN, H_DIM), lambda i: (i, 0)),
    pl.BlockSpec((BN, 16), lambda i: (i, 0)),
    pl.BlockSpec((BN, 16), lambda i: (i, 0)),
    pl.BlockSpec((1, 16), lambda i: (0, 0)),
]

_W_SPEC = pl.BlockSpec((H_DIM, H_DIM), lambda i: (0, 0))
_ASER_SPEC = pl.BlockSpec((H_DIM, 32), lambda i: (0, 0))
_ROW_SPEC = pl.BlockSpec((BN, H_DIM), lambda i: (i, 0))
_COL_SPEC = pl.BlockSpec((BN, 1), lambda i: (i, 0))


def _al_call(x_prev, o0, o1, snorm, W, aser):
    return pl.pallas_call(
        _al_body,
        grid=(GRID,),
        in_specs=[_ROW_SPEC, _ROW_SPEC, _ROW_SPEC, _COL_SPEC, _W_SPEC, _ASER_SPEC],
        out_specs=_A_OUT_SPECS,
        out_shape=_A_OUT_SHAPES,
    )(x_prev, o0, o1, snorm, W, aser)


def _ro_body(xp_ref, o0_ref, o1_ref, sn_ref, gid_ref, wro_ref, wp_ref, bp_ref,
             out_ref, sums_ref, cnts_ref):
    i = pl.program_id(0)
    z = (o0_ref[...] + o1_ref[...]) * sn_ref[...]
    z = jnp.where(z > 0, z, jnp.exp(z) - 1.0)
    y = xp_ref[...] + z
    hn = jnp.dot(y, wro_ref[...], preferred_element_type=_f32)
    gid = gid_ref[...]
    iota = lax.broadcasted_iota(jnp.int32, (BN, G), 1)
    p = (gid == iota).astype(_f32)
    psum = lax.dot_general(p, hn, (((0,), (0,)), ((), ())),
                           preferred_element_type=_f32)
    ones = jnp.ones((BN, H_DIM), _f32)
    pcnt = lax.dot_general(p, ones, (((0,), (0,)), ((), ())),
                           preferred_element_type=_f32)

    @pl.when(i == 0)
    def _():
        sums_ref[...] = psum
        cnts_ref[...] = pcnt

    @pl.when(i > 0)
    def _():
        sums_ref[...] = sums_ref[...] + psum
        cnts_ref[...] = cnts_ref[...] + pcnt

    @pl.when(i == GRID - 1)
    def _():
        hg = sums_ref[...] / jnp.maximum(cnts_ref[...], 1.0)
        out_ref[...] = jnp.dot(hg, wp_ref[...], preferred_element_type=_f32) + bp_ref[...]


def _ro_call(x_prev, o0, o1, snorm, gid2d, W_ro, W_pred, b_pred2d):
    return pl.pallas_call(
        _ro_body,
        grid=(GRID,),
        in_specs=[
            _ROW_SPEC, _ROW_SPEC, _ROW_SPEC, _COL_SPEC,
            pl.BlockSpec((BN, 1), lambda i: (i, 0)),
            pl.BlockSpec((H_DIM, H_DIM), lambda i: (0, 0)),
            pl.BlockSpec((H_DIM, 1), lambda i: (0, 0)),
            pl.BlockSpec((1, 1), lambda i: (0, 0)),
        ],
        out_specs=pl.BlockSpec((G, 1), lambda i: (0, 0)),
        out_shape=jax.ShapeDtypeStruct((G, 1), _f32),
        scratch_shapes=[
            pltpu.VMEM((G, H_DIM), _f32),
            pltpu.VMEM((G, H_DIM), _f32),
        ],
    )(x_prev, o0, o1, snorm, gid2d, W_ro, W_pred, b_pred2d)


def _r_body(d0_ref, d1_ref, r_ref):
    r_ref[...] = 1.0 / (d0_ref[...] + d1_ref[...] + 1e-9)


def _r_call(den0, den1):
    return pl.pallas_call(
        _r_body,
        grid=(GRID,),
        in_specs=[pl.BlockSpec((BN, 16), lambda i: (i, 0)),
                  pl.BlockSpec((BN, 16), lambda i: (i, 0))],
        out_specs=pl.BlockSpec((BN, 16), lambda i: (i, 0)),
        out_shape=jax.ShapeDtypeStruct((N, 16), _f32),
    )(den0, den1)


_MESH = plsc.VectorSubcoreMesh(core_axis_name="c", subcore_axis_name="s")


@functools.partial(
    pl.kernel,
    out_type=[
        jax.ShapeDtypeStruct((E, 16), _f32),
        jax.ShapeDtypeStruct((N, 16), _f32),
        jax.ShapeDtypeStruct((N, 16), _f32),
    ],
    mesh=_MESH,
    compiler_params=pltpu.CompilerParams(use_tc_tiling_on_sc=False),
    scratch_types=[
        pltpu.VMEM((CK_E,), jnp.int32),
        pltpu.VMEM((CK_E,), jnp.int32),
        pltpu.VMEM((CK_E, 16), _f32),
        pltpu.VMEM((CK_E, 16), _f32),
        pltpu.VMEM((CK_E, 16), _f32),
        pltpu.VMEM((16,), _f32),
        pltpu.VMEM_SHARED((N, 16), _f32),
        pltpu.SemaphoreType.DMA,
        pltpu.SemaphoreType.DMA,
    ],
)
def _e_kernel(src_hbm, dst_hbm, el_hbm, er_hbm, m_hbm, z16_hbm,
              w_hbm, den0_hbm, den1_hbm,
              idx_s, idx_d, abuf, bbuf, wbuf, mvec, den_sh,
              sem1, sem2):
    c = lax.axis_index("c")
    s = lax.axis_index("s")
    wid = s * NC + c

    pltpu.sync_copy(z16_hbm.at[pl.ds(s * RPS, RPS)], den_sh.at[pl.ds(s * RPS, RPS)])

    @pl.when(s == NS - 1)
    def _():
        pltpu.sync_copy(z16_hbm.at[pl.ds(NS * RPS, TAIL)],
                        den_sh.at[pl.ds(NS * RPS, TAIL)])

    pltpu.sync_copy(m_hbm, mvec)
    plsc.subcore_barrier()

    mv = mvec[...]

    def chunk(j, carry):
        base = wid * EPW + j * CK_E
        pltpu.sync_copy(src_hbm.at[pl.ds(base, CK_E)], idx_s)
        pltpu.sync_copy(dst_hbm.at[pl.ds(base, CK_E)], idx_d)
        cp1 = pltpu.async_copy(el_hbm.at[idx_s], abuf, sem1)
        cp2 = pltpu.async_copy(er_hbm.at[idx_d], bbuf, sem2)
        cp1.wait()
        cp2.wait()

        @plsc.parallel_loop(0, CK_E, unroll=8)
        def vbody(i):
            a = abuf[i, :]
            b = bbuf[i, :]
            e = a + b
            e = jnp.maximum(e, 0.2 * e)
            cb = mv + b
            cb = jnp.maximum(cb, 0.2 * cb)
            wbuf[i, :] = jnp.exp(e - cb)
        pltpu.sync_copy(wbuf, w_hbm.at[pl.ds(base, CK_E)])
        pltpu.sync_copy(wbuf, den_sh.at[idx_d], add=True)
        return carry

    lax.fori_loop(0, NIT_E, chunk, 0)

    plsc.subcore_barrier()

    @pl.when(c == 0)
    def _():
        pltpu.sync_copy(den_sh.at[pl.ds(s * RPS, RPS)], den0_hbm.at[pl.ds(s * RPS, RPS)])

        @pl.when(s == NS - 1)
        def _():
            pltpu.sync_copy(den_sh.at[pl.ds(NS * RPS, TAIL)],
                            den0_hbm.at[pl.ds(NS * RPS, TAIL)])

    @pl.when(c == 1)
    def _():
        pltpu.sync_copy(den_sh.at[pl.ds(s * RPS, RPS)], den1_hbm.at[pl.ds(s * RPS, RPS)])

        @pl.when(s == NS - 1)
        def _():
            pltpu.sync_copy(den_sh.at[pl.ds(NS * RPS, TAIL)],
                            den1_hbm.at[pl.ds(NS * RPS, TAIL)])


@functools.partial(
    pl.kernel,
    out_type=[
        jax.ShapeDtypeStruct((N, H_DIM), _f32),
        jax.ShapeDtypeStruct((N, H_DIM), _f32),
    ],
    mesh=_MESH,
    compiler_params=pltpu.CompilerParams(use_tc_tiling_on_sc=False),
    scratch_types=[
        pltpu.VMEM((NIT_M, CK_M), jnp.int32),
        pltpu.VMEM((NIT_M, CK_M), jnp.int32),
        pltpu.VMEM((CK_M, H_DIM), _f32),
        pltpu.VMEM((CK_M, H_DIM), _f32),
        pltpu.VMEM((CK_M, 16), _f32),
        pltpu.VMEM((CK_M, 16), _f32),
        pltpu.VMEM_SHARED((N, H_DIM), _f32),
        pltpu.SemaphoreType.DMA,
        pltpu.SemaphoreType.DMA,
    ],
)
def _m_kernel(src3_hbm, dst3_hbm, wh_hbm, w_hbm, r_hbm, z128_hbm,
              o0_hbm, o1_hbm,
              idx_s2d, idx_d2d, gb0, gb1, wb, rg, out_sh,
              semg, semsc):
    c = lax.axis_index("c")
    s = lax.axis_index("s")
    wid = s * NC + c

    pltpu.sync_copy(z128_hbm.at[pl.ds(s * RPS, RPS)], out_sh.at[pl.ds(s * RPS, RPS)])

    @pl.when(s == NS - 1)
    def _():
        pltpu.sync_copy(z128_hbm.at[pl.ds(NS * RPS, TAIL)],
                        out_sh.at[pl.ds(NS * RPS, TAIL)])

    pltpu.sync_copy(src3_hbm.at[wid], idx_s2d)
    pltpu.sync_copy(dst3_hbm.at[wid], idx_d2d)
    plsc.subcore_barrier()

    def drain_scatter(gb):
        pltpu.make_async_copy(gb, out_sh.at[idx_d2d.at[0]], semsc).wait()

    def do_chunk(j, gb):
        cp1 = pltpu.async_copy(wh_hbm.at[idx_s2d.at[j]], gb, semg)
        cp2 = pltpu.async_copy(r_hbm.at[idx_d2d.at[j]], rg, semg)
        pltpu.sync_copy(w_hbm.at[pl.ds(wid * EPW + j * CK_M, CK_M)], wb)
        cp1.wait()
        cp2.wait()

        @plsc.parallel_loop(0, CK_M, unroll=4)
        def vbody(i):
            aw = wb[i, :] * rg[i, :]
            for hd in range(HEADS):
                gb[i, 16 * hd:16 * (hd + 1)] = (
                    gb[i, 16 * hd:16 * (hd + 1)] * aw[hd])

        pltpu.async_copy(gb, out_sh.at[idx_d2d.at[j]], semsc, add=True)

    def pair(t, carry):
        @pl.when(t > 0)
        def _():
            drain_scatter(gb0)

        do_chunk(2 * t, gb0)

        @pl.when(t > 0)
        def _():
            drain_scatter(gb1)

        do_chunk(2 * t + 1, gb1)
        return carry

    lax.fori_loop(0, NIT_M // 2, pair, 0)
    drain_scatter(gb0)
    do_chunk(NIT_M - 1, gb0)
    drain_scatter(gb1)
    drain_scatter(gb0)

    plsc.subcore_barrier()

    @pl.when(c == 0)
    def _():
        pltpu.sync_copy(out_sh.at[pl.ds(s * RPS, RPS)], o0_hbm.at[pl.ds(s * RPS, RPS)])

        @pl.when(s == NS - 1)
        def _():
            pltpu.sync_copy(out_sh.at[pl.ds(NS * RPS, TAIL)],
                            o0_hbm.at[pl.ds(NS * RPS, TAIL)])

    @pl.when(c == 1)
    def _():
        pltpu.sync_copy(out_sh.at[pl.ds(s * RPS, RPS)], o1_hbm.at[pl.ds(s * RPS, RPS)])

        @pl.when(s == NS - 1)
        def _():
            pltpu.sync_copy(out_sh.at[pl.ds(NS * RPS, TAIL)],
                            o1_hbm.at[pl.ds(NS * RPS, TAIL)])


def kernel(h, edge_index, node_graph_id, snorm_n, W_embed, Ws, a_srcs, a_dsts,
           W_ro, W_pred, b_pred):
    src = edge_index[0]
    dst = edge_index[1]

    head_of = jnp.arange(H_DIM, dtype=jnp.int32) // DH
    delta = (head_of[:, None] == jnp.arange(HEADS)[None, :]).astype(_f32)
    asv = a_srcs.reshape(L, H_DIM)[:, :, None] * delta[None]
    adv = a_dsts.reshape(L, H_DIM)[:, :, None] * delta[None]
    aser = jnp.concatenate([asv, asv, adv, adv], axis=2)

    zeros16 = jnp.zeros((N, 16), _f32)
    zeros128 = jnp.zeros((N, H_DIM), _f32)
    snorm = snorm_n.astype(_f32)
    src3 = src.reshape(NW, NIT_M, CK_M)
    dst3 = dst.reshape(NW, NIT_M, CK_M)

    x0 = _x0_call(h.astype(_f32), W_embed)

    def layer(l, carry):
        x, o0, o1 = carry
        W_l = lax.dynamic_index_in_dim(Ws, l, 0, keepdims=False)
        aser_l = lax.dynamic_index_in_dim(aser, l, 0, keepdims=False)
        x, wh, el, er, m = _al_call(x, o0, o1, snorm, W_l, aser_l)
        w, den0, den1 = _e_kernel(src, dst, el, er, m.reshape(16), zeros16)
        r = _r_call(den0, den1)
        o0, o1 = _m_kernel(src3, dst3, wh, w, r, zeros128)
        return (x, o0, o1)

    x, o0, o1 = lax.fori_loop(0, L, layer, (x0, zeros128, zeros128))

    gid2d = node_graph_id.reshape(N, 1)
    b2d = b_pred.reshape(1, 1)
    return _ro_call(x, o0, o1, snorm, gid2d, W_ro, W_pred, b2d)


# speedup vs baseline: 132.1437x; 1.1858x over previous
import functools

import jax
import jax.numpy as jnp
from jax import lax
from jax.experimental import pallas as pl
from jax.experimental.pallas import tpu as pltpu
from jax.experimental.pallas import tpu_sc as plsc

N = 10000
E = 320000
H_DIM = 128
HEADS = 8
DH = 16
L = 4
G = 64

NC = 2
NS = 16
NW = NC * NS
EPW = E // NW
RPS = 624
TAIL = N - NS * RPS

BN = 2000
GRID = N // BN

CK_E = 2000
NIT_E = EPW // CK_E
CK_M = 80
NIT_M = EPW // CK_M

_f32 = jnp.float32


def _x0_body(h_ref, we_ref, x_ref):
    x_ref[...] = jnp.dot(h_ref[...], we_ref[...], preferred_element_type=_f32)


def _x0_call(h, W_embed):
    return pl.pallas_call(
        _x0_body,
        grid=(GRID,),
        in_specs=[_ROW_SPEC, _W_SPEC],
        out_specs=_ROW_SPEC,
        out_shape=jax.ShapeDtypeStruct((N, H_DIM), _f32),
    )(h, W_embed)


def _al_body(xp_ref, o0_ref, o1_ref, sn_ref, w_ref, aser_ref,
             x_ref, wh_ref, el_ref, er_ref, m_ref):
    i = pl.program_id(0)
    z = (o0_ref[...] + o1_ref[...]) * sn_ref[...]
    z = jnp.where(z > 0, z, jnp.exp(z) - 1.0)
    x = xp_ref[...] + z
    x_ref[...] = x
    wh = jnp.dot(x, w_ref[...], preferred_element_type=_f32)
    wh_ref[...] = wh
    eler = jnp.dot(wh, aser_ref[...], preferred_element_type=_f32)
    el_ref[...] = eler[:, :16]
    er_ref[...] = eler[:, 16:]
    blkmax = jnp.max(eler[:, :16], axis=0, keepdims=True)

    @pl.when(i == 0)
    def _():
        m_ref[...] = blkmax

    @pl.when(i > 0)
    def _():
        m_ref[...] = jnp.maximum(m_ref[...], blkmax)


_A_OUT_SHAPES = [
    jax.ShapeDtypeStruct((N, H_DIM), _f32),
    jax.ShapeDtypeStruct((N, H_DIM), _f32),
    jax.ShapeDtypeStruct((N, 16), _f32),
    jax.ShapeDtypeStruct((N, 16), _f32),
    jax.ShapeDtypeStruct((1, 16), _f32),
]

_A_OUT_SPECS = [
    pl.BlockSpec((BN, H_DIM), lambda i: (i, 0)),
    pl.BlockSpec((BN, H_DIM), lambda i: (i, 0)),
    pl.BlockSpec((BN, 16), lambda i: (i, 0)),
    pl.BlockSpec((BN, 16), lambda i: (i, 0)),
    pl.BlockSpec((1, 16), lambda i: (0, 0)),
]

_W_SPEC = pl.BlockSpec((H_DIM, H_DIM), lambda i: (0, 0))
_ASER_SPEC = pl.BlockSpec((H_DIM, 32), lambda i: (0, 0))
_ROW_SPEC = pl.BlockSpec((BN, H_DIM), lambda i: (i, 0))
_COL_SPEC = pl.BlockSpec((BN, 1), lambda i: (i, 0))


def _al_call(x_prev, o0, o1, snorm, W, aser):
    return pl.pallas_call(
        _al_body,
        grid=(GRID,),
        in_specs=[_ROW_SPEC, _ROW_SPEC, _ROW_SPEC, _COL_SPEC, _W_SPEC, _ASER_SPEC],
        out_specs=_A_OUT_SPECS,
        out_shape=_A_OUT_SHAPES,
    )(x_prev, o0, o1, snorm, W, aser)


def _ro_body(xp_ref, o0_ref, o1_ref, sn_ref, gid_ref, wro_ref, wp_ref, bp_ref,
             out_ref, sums_ref, cnts_ref):
    i = pl.program_id(0)
    z = (o0_ref[...] + o1_ref[...]) * sn_ref[...]
    z = jnp.where(z > 0, z, jnp.exp(z) - 1.0)
    y = xp_ref[...] + z
    hn = jnp.dot(y, wro_ref[...], preferred_element_type=_f32)
    gid = gid_ref[...]
    iota = lax.broadcasted_iota(jnp.int32, (BN, G), 1)
    p = (gid == iota).astype(_f32)
    psum = lax.dot_general(p, hn, (((0,), (0,)), ((), ())),
                           preferred_element_type=_f32)
    ones = jnp.ones((BN, H_DIM), _f32)
    pcnt = lax.dot_general(p, ones, (((0,), (0,)), ((), ())),
                           preferred_element_type=_f32)

    @pl.when(i == 0)
    def _():
        sums_ref[...] = psum
        cnts_ref[...] = pcnt

    @pl.when(i > 0)
    def _():
        sums_ref[...] = sums_ref[...] + psum
        cnts_ref[...] = cnts_ref[...] + pcnt

    @pl.when(i == GRID - 1)
    def _():
        hg = sums_ref[...] / jnp.maximum(cnts_ref[...], 1.0)
        out_ref[...] = jnp.dot(hg, wp_ref[...], preferred_element_type=_f32) + bp_ref[...]


def _ro_call(x_prev, o0, o1, snorm, gid2d, W_ro, W_pred, b_pred2d):
    return pl.pallas_call(
        _ro_body,
        grid=(GRID,),
        in_specs=[
            _ROW_SPEC, _ROW_SPEC, _ROW_SPEC, _COL_SPEC,
            pl.BlockSpec((BN, 1), lambda i: (i, 0)),
            pl.BlockSpec((H_DIM, H_DIM), lambda i: (0, 0)),
            pl.BlockSpec((H_DIM, 1), lambda i: (0, 0)),
            pl.BlockSpec((1, 1), lambda i: (0, 0)),
        ],
        out_specs=pl.BlockSpec((G, 1), lambda i: (0, 0)),
        out_shape=jax.ShapeDtypeStruct((G, 1), _f32),
        scratch_shapes=[
            pltpu.VMEM((G, H_DIM), _f32),
            pltpu.VMEM((G, H_DIM), _f32),
        ],
    )(x_prev, o0, o1, snorm, gid2d, W_ro, W_pred, b_pred2d)


def _r_body(d0_ref, d1_ref, r_ref):
    r_ref[...] = 1.0 / (d0_ref[...] + d1_ref[...] + 1e-9)


def _r_call(den0, den1):
    return pl.pallas_call(
        _r_body,
        grid=(GRID,),
        in_specs=[pl.BlockSpec((BN, 16), lambda i: (i, 0)),
                  pl.BlockSpec((BN, 16), lambda i: (i, 0))],
        out_specs=pl.BlockSpec((BN, 16), lambda i: (i, 0)),
        out_shape=jax.ShapeDtypeStruct((N, 16), _f32),
    )(den0, den1)


_MESH = plsc.VectorSubcoreMesh(core_axis_name="c", subcore_axis_name="s")


@functools.partial(
    pl.kernel,
    out_type=[
        jax.ShapeDtypeStruct((E, 16), _f32),
        jax.ShapeDtypeStruct((N, 16), _f32),
        jax.ShapeDtypeStruct((N, 16), _f32),
    ],
    mesh=_MESH,
    compiler_params=pltpu.CompilerParams(use_tc_tiling_on_sc=False),
    scratch_types=[
        pltpu.VMEM((CK_E,), jnp.int32),
        pltpu.VMEM((CK_E,), jnp.int32),
        pltpu.VMEM((CK_E, 16), _f32),
        pltpu.VMEM((CK_E, 16), _f32),
        pltpu.VMEM((CK_E, 16), _f32),
        pltpu.VMEM((16,), _f32),
        pltpu.VMEM_SHARED((N, 16), _f32),
        pltpu.SemaphoreType.DMA,
        pltpu.SemaphoreType.DMA,
    ],
)
def _e_kernel(src_hbm, dst_hbm, el_hbm, er_hbm, m_hbm, z16_hbm,
              w_hbm, den0_hbm, den1_hbm,
              idx_s, idx_d, abuf, bbuf, wbuf, mvec, den_sh,
              sem1, sem2):
    c = lax.axis_index("c")
    s = lax.axis_index("s")
    wid = s * NC + c

    pltpu.sync_copy(z16_hbm.at[pl.ds(s * RPS, RPS)], den_sh.at[pl.ds(s * RPS, RPS)])

    @pl.when(s == NS - 1)
    def _():
        pltpu.sync_copy(z16_hbm.at[pl.ds(NS * RPS, TAIL)],
                        den_sh.at[pl.ds(NS * RPS, TAIL)])

    pltpu.sync_copy(m_hbm, mvec)
    plsc.subcore_barrier()

    mv = mvec[...]

    def chunk(j, carry):
        base = wid * EPW + j * CK_E
        pltpu.sync_copy(src_hbm.at[pl.ds(base, CK_E)], idx_s)
        pltpu.sync_copy(dst_hbm.at[pl.ds(base, CK_E)], idx_d)
        cp1 = pltpu.async_copy(el_hbm.at[idx_s], abuf, sem1)
        cp2 = pltpu.async_copy(er_hbm.at[idx_d], bbuf, sem2)
        cp1.wait()
        cp2.wait()

        @plsc.parallel_loop(0, CK_E, unroll=8)
        def vbody(i):
            a = abuf[i, :]
            b = bbuf[i, :]
            e = a + b
            e = jnp.maximum(e, 0.2 * e)
            cb = mv + b
            cb = jnp.maximum(cb, 0.2 * cb)
            wbuf[i, :] = jnp.exp(e - cb)
        pltpu.sync_copy(wbuf, w_hbm.at[pl.ds(base, CK_E)])
        pltpu.sync_copy(wbuf, den_sh.at[idx_d], add=True)
        return carry

    lax.fori_loop(0, NIT_E, chunk, 0)

    plsc.subcore_barrier()

    @pl.when(c == 0)
    def _():
        pltpu.sync_copy(den_sh.at[pl.ds(s * RPS, RPS)], den0_hbm.at[pl.ds(s * RPS, RPS)])

        @pl.when(s == NS - 1)
        def _():
            pltpu.sync_copy(den_sh.at[pl.ds(NS * RPS, TAIL)],
                            den0_hbm.at[pl.ds(NS * RPS, TAIL)])

    @pl.when(c == 1)
    def _():
        pltpu.sync_copy(den_sh.at[pl.ds(s * RPS, RPS)], den1_hbm.at[pl.ds(s * RPS, RPS)])

        @pl.when(s == NS - 1)
        def _():
            pltpu.sync_copy(den_sh.at[pl.ds(NS * RPS, TAIL)],
                            den1_hbm.at[pl.ds(NS * RPS, TAIL)])


@functools.partial(
    pl.kernel,
    out_type=[
        jax.ShapeDtypeStruct((N, H_DIM), _f32),
        jax.ShapeDtypeStruct((N, H_DIM), _f32),
    ],
    mesh=_MESH,
    compiler_params=pltpu.CompilerParams(use_tc_tiling_on_sc=False),
    scratch_types=[
        pltpu.VMEM((NIT_M, CK_M), jnp.int32),
        pltpu.VMEM((NIT_M, CK_M), jnp.int32),
        pltpu.VMEM((CK_M, H_DIM), _f32),
        pltpu.VMEM((CK_M, H_DIM), _f32),
        pltpu.VMEM((CK_M, 16), _f32),
        pltpu.VMEM((CK_M, 16), _f32),
        pltpu.VMEM((CK_M, 16), _f32),
        pltpu.VMEM((CK_M, 16), _f32),
        pltpu.VMEM_SHARED((N, H_DIM), _f32),
        pltpu.SemaphoreType.DMA,
        pltpu.SemaphoreType.DMA,
        pltpu.SemaphoreType.DMA,
        pltpu.SemaphoreType.DMA,
    ],
)
def _m_kernel(src3_hbm, dst3_hbm, wh_hbm, w_hbm, r_hbm, z128_hbm,
              o0_hbm, o1_hbm,
              idx_s2d, idx_d2d, gb0, gb1, wb0, wb1, rg0, rg1, out_sh,
              sem_wh, sem_r, sem_w, semsc):
    c = lax.axis_index("c")
    s = lax.axis_index("s")
    wid = s * NC + c

    pltpu.sync_copy(z128_hbm.at[pl.ds(s * RPS, RPS)], out_sh.at[pl.ds(s * RPS, RPS)])

    @pl.when(s == NS - 1)
    def _():
        pltpu.sync_copy(z128_hbm.at[pl.ds(NS * RPS, TAIL)],
                        out_sh.at[pl.ds(NS * RPS, TAIL)])

    pltpu.sync_copy(src3_hbm.at[wid], idx_s2d)
    pltpu.sync_copy(dst3_hbm.at[wid], idx_d2d)
    plsc.subcore_barrier()

    def issue_gathers(j, gb, wbx, rgx):
        pltpu.async_copy(wh_hbm.at[idx_s2d.at[j]], gb, sem_wh)
        pltpu.async_copy(r_hbm.at[idx_d2d.at[j]], rgx, sem_r)
        pltpu.async_copy(w_hbm.at[pl.ds(wid * EPW + j * CK_M, CK_M)], wbx, sem_w)

    def drain_gathers(gb, wbx, rgx):
        pltpu.make_async_copy(wh_hbm.at[idx_s2d.at[0]], gb, sem_wh).wait()
        pltpu.make_async_copy(r_hbm.at[idx_d2d.at[0]], rgx, sem_r).wait()
        pltpu.make_async_copy(w_hbm.at[pl.ds(0, CK_M)], wbx, sem_w).wait()

    def drain_scatter():
        pltpu.make_async_copy(gb0, out_sh.at[idx_d2d.at[0]], semsc).wait()

    def compute_and_scatter(j, gb, wbx, rgx):
        @plsc.parallel_loop(0, CK_M, unroll=4)
        def vbody(i):
            aw = wbx[i, :] * rgx[i, :]
            for hd in range(HEADS):
                gb[i, 16 * hd:16 * (hd + 1)] = (
                    gb[i, 16 * hd:16 * (hd + 1)] * aw[hd])

        pltpu.async_copy(gb, out_sh.at[idx_d2d.at[j]], semsc, add=True)

    issue_gathers(0, gb0, wb0, rg0)

    def pair(t, carry):
        j0 = 2 * t
        drain_gathers(gb0, wb0, rg0)

        @pl.when(t > 0)
        def _():
            drain_scatter()

        issue_gathers(j0 + 1, gb1, wb1, rg1)
        compute_and_scatter(j0, gb0, wb0, rg0)

        drain_gathers(gb1, wb1, rg1)
        drain_scatter()
        issue_gathers(j0 + 2, gb0, wb0, rg0)
        compute_and_scatter(j0 + 1, gb1, wb1, rg1)
        return carry

    lax.fori_loop(0, NIT_M // 2, pair, 0)
    drain_gathers(gb0, wb0, rg0)
    drain_scatter()
    compute_and_scatter(NIT_M - 1, gb0, wb0, rg0)
    drain_scatter()
    plsc.subcore_barrier()

    @pl.when(c == 0)
    def _():
        pltpu.sync_copy(out_sh.at[pl.ds(s * RPS, RPS)], o0_hbm.at[pl.ds(s * RPS, RPS)])

        @pl.when(s == NS - 1)
        def _():
            pltpu.sync_copy(out_sh.at[pl.ds(NS * RPS, TAIL)],
                            o0_hbm.at[pl.ds(NS * RPS, TAIL)])

    @pl.when(c == 1)
    def _():
        pltpu.sync_copy(out_sh.at[pl.ds(s * RPS, RPS)], o1_hbm.at[pl.ds(s * RPS, RPS)])

        @pl.when(s == NS - 1)
        def _():
            pltpu.sync_copy(out_sh.at[pl.ds(NS * RPS, TAIL)],
                            o1_hbm.at[pl.ds(NS * RPS, TAIL)])


def kernel(h, edge_index, node_graph_id, snorm_n, W_embed, Ws, a_srcs, a_dsts,
           W_ro, W_pred, b_pred):
    src = edge_index[0]
    dst = edge_index[1]

    head_of = jnp.arange(H_DIM, dtype=jnp.int32) // DH
    delta = (head_of[:, None] == jnp.arange(HEADS)[None, :]).astype(_f32)
    asv = a_srcs.reshape(L, H_DIM)[:, :, None] * delta[None]
    adv = a_dsts.reshape(L, H_DIM)[:, :, None] * delta[None]
    aser = jnp.concatenate([asv, asv, adv, adv], axis=2)

    zeros16 = jnp.zeros((N, 16), _f32)
    zeros128 = jnp.zeros((N, H_DIM), _f32)
    snorm = snorm_n.astype(_f32)
    src3 = src.reshape(NW, NIT_M, CK_M)
    dst3 = dst.reshape(NW, NIT_M, CK_M)

    x0 = _x0_call(h.astype(_f32), W_embed)

    def layer(l, carry):
        x, o0, o1 = carry
        W_l = lax.dynamic_index_in_dim(Ws, l, 0, keepdims=False)
        aser_l = lax.dynamic_index_in_dim(aser, l, 0, keepdims=False)
        x, wh, el, er, m = _al_call(x, o0, o1, snorm, W_l, aser_l)
        w, den0, den1 = _e_kernel(src, dst, el, er, m.reshape(16), zeros16)
        r = _r_call(den0, den1)
        o0, o1 = _m_kernel(src3, dst3, wh, w, r, zeros128)
        return (x, o0, o1)

    x, o0, o1 = lax.fori_loop(0, L, layer, (x0, zeros128, zeros128))

    gid2d = node_graph_id.reshape(N, 1)
    b2d = b_pred.reshape(1, 1)
    return _ro_call(x, o0, o1, snorm, gid2d, W_ro, W_pred, b2d)


# speedup vs baseline: 143.0310x; 1.0824x over previous
import functools

import jax
import jax.numpy as jnp
from jax import lax
from jax.experimental import pallas as pl
from jax.experimental.pallas import tpu as pltpu
from jax.experimental.pallas import tpu_sc as plsc

N = 10000
E = 320000
H_DIM = 128
HEADS = 8
DH = 16
L = 4
G = 64

NC = 2
NS = 16
NW = NC * NS
EPW = E // NW
RPS = 624
TAIL = N - NS * RPS

BN = 2000
GRID = N // BN

CK_E = 1000
NIT_E = EPW // CK_E
CK_M = 80
NIT_M = EPW // CK_M

_f32 = jnp.float32


def _x0_body(h_ref, we_ref, x_ref):
    x_ref[...] = jnp.dot(h_ref[...], we_ref[...], preferred_element_type=_f32)


def _x0_call(h, W_embed):
    return pl.pallas_call(
        _x0_body,
        grid=(GRID,),
        in_specs=[_ROW_SPEC, _W_SPEC],
        out_specs=_ROW_SPEC,
        out_shape=jax.ShapeDtypeStruct((N, H_DIM), _f32),
    )(h, W_embed)


def _al_body(xp_ref, o0_ref, o1_ref, sn_ref, w_ref, aser_ref,
             x_ref, wh_ref, el_ref, er_ref, m_ref):
    i = pl.program_id(0)
    z = (o0_ref[...] + o1_ref[...]) * sn_ref[...]
    z = jnp.where(z > 0, z, jnp.exp(z) - 1.0)
    x = xp_ref[...] + z
    x_ref[...] = x
    wh = jnp.dot(x, w_ref[...], preferred_element_type=_f32)
    wh_ref[...] = wh
    eler = jnp.dot(wh, aser_ref[...], preferred_element_type=_f32)
    el_ref[...] = eler[:, :16]
    er_ref[...] = eler[:, 16:]
    blkmax = jnp.max(eler[:, :16], axis=0, keepdims=True)

    @pl.when(i == 0)
    def _():
        m_ref[...] = blkmax

    @pl.when(i > 0)
    def _():
        m_ref[...] = jnp.maximum(m_ref[...], blkmax)


_A_OUT_SHAPES = [
    jax.ShapeDtypeStruct((N, H_DIM), _f32),
    jax.ShapeDtypeStruct((N, H_DIM), _f32),
    jax.ShapeDtypeStruct((N, 16), _f32),
    jax.ShapeDtypeStruct((N, 16), _f32),
    jax.ShapeDtypeStruct((1, 16), _f32),
]

_A_OUT_SPECS = [
    pl.BlockSpec((BN, H_DIM), lambda i: (i, 0)),
    pl.BlockSpec((BN, H_DIM), lambda i: (i, 0)),
    pl.BlockSpec((BN, 16), lambda i: (i, 0)),
    pl.BlockSpec((BN, 16), lambda i: (i, 0)),
    pl.BlockSpec((1, 16), lambda i: (0, 0)),
]

_W_SPEC = pl.BlockSpec((H_DIM, H_DIM), lambda i: (0, 0))
_ASER_SPEC = pl.BlockSpec((H_DIM, 32), lambda i: (0, 0))
_ROW_SPEC = pl.BlockSpec((BN, H_DIM), lambda i: (i, 0))
_COL_SPEC = pl.BlockSpec((BN, 1), lambda i: (i, 0))


def _al_call(x_prev, o0, o1, snorm, W, aser):
    return pl.pallas_call(
        _al_body,
        grid=(GRID,),
        in_specs=[_ROW_SPEC, _ROW_SPEC, _ROW_SPEC, _COL_SPEC, _W_SPEC, _ASER_SPEC],
        out_specs=_A_OUT_SPECS,
        out_shape=_A_OUT_SHAPES,
    )(x_prev, o0, o1, snorm, W, aser)


def _ro_body(xp_ref, o0_ref, o1_ref, sn_ref, gid_ref, wro_ref, wp_ref, bp_ref,
             out_ref, sums_ref, cnts_ref):
    i = pl.program_id(0)
    z = (o0_ref[...] + o1_ref[...]) * sn_ref[...]
    z = jnp.where(z > 0, z, jnp.exp(z) - 1.0)
    y = xp_ref[...] + z
    hn = jnp.dot(y, wro_ref[...], preferred_element_type=_f32)
    gid = gid_ref[...]
    iota = lax.broadcasted_iota(jnp.int32, (BN, G), 1)
    p = (gid == iota).astype(_f32)
    psum = lax.dot_general(p, hn, (((0,), (0,)), ((), ())),
                           preferred_element_type=_f32)
    ones = jnp.ones((BN, H_DIM), _f32)
    pcnt = lax.dot_general(p, ones, (((0,), (0,)), ((), ())),
                           preferred_element_type=_f32)

    @pl.when(i == 0)
    def _():
        sums_ref[...] = psum
        cnts_ref[...] = pcnt

    @pl.when(i > 0)
    def _():
        sums_ref[...] = sums_ref[...] + psum
        cnts_ref[...] = cnts_ref[...] + pcnt

    @pl.when(i == GRID - 1)
    def _():
        hg = sums_ref[...] / jnp.maximum(cnts_ref[...], 1.0)
        out_ref[...] = jnp.dot(hg, wp_ref[...], preferred_element_type=_f32) + bp_ref[...]


def _ro_call(x_prev, o0, o1, snorm, gid2d, W_ro, W_pred, b_pred2d):
    return pl.pallas_call(
        _ro_body,
        grid=(GRID,),
        in_specs=[
            _ROW_SPEC, _ROW_SPEC, _ROW_SPEC, _COL_SPEC,
            pl.BlockSpec((BN, 1), lambda i: (i, 0)),
            pl.BlockSpec((H_DIM, H_DIM), lambda i: (0, 0)),
            pl.BlockSpec((H_DIM, 1), lambda i: (0, 0)),
            pl.BlockSpec((1, 1), lambda i: (0, 0)),
        ],
        out_specs=pl.BlockSpec((G, 1), lambda i: (0, 0)),
        out_shape=jax.ShapeDtypeStruct((G, 1), _f32),
        scratch_shapes=[
            pltpu.VMEM((G, H_DIM), _f32),
            pltpu.VMEM((G, H_DIM), _f32),
        ],
    )(x_prev, o0, o1, snorm, gid2d, W_ro, W_pred, b_pred2d)


def _r_body(d0_ref, d1_ref, r_ref):
    r_ref[...] = 1.0 / (d0_ref[...] + d1_ref[...] + 1e-9)


def _r_call(den0, den1):
    return pl.pallas_call(
        _r_body,
        grid=(GRID,),
        in_specs=[pl.BlockSpec((BN, 16), lambda i: (i, 0)),
                  pl.BlockSpec((BN, 16), lambda i: (i, 0))],
        out_specs=pl.BlockSpec((BN, 16), lambda i: (i, 0)),
        out_shape=jax.ShapeDtypeStruct((N, 16), _f32),
    )(den0, den1)


_MESH = plsc.VectorSubcoreMesh(core_axis_name="c", subcore_axis_name="s")


@functools.partial(
    pl.kernel,
    out_type=[
        jax.ShapeDtypeStruct((E, 16), _f32),
        jax.ShapeDtypeStruct((N, 16), _f32),
        jax.ShapeDtypeStruct((N, 16), _f32),
    ],
    mesh=_MESH,
    compiler_params=pltpu.CompilerParams(use_tc_tiling_on_sc=False),
    scratch_types=[
        pltpu.VMEM((NIT_E, CK_E), jnp.int32),
        pltpu.VMEM((NIT_E, CK_E), jnp.int32),
        pltpu.VMEM((CK_E, 16), _f32),
        pltpu.VMEM((CK_E, 16), _f32),
        pltpu.VMEM((CK_E, 16), _f32),
        pltpu.VMEM((CK_E, 16), _f32),
        pltpu.VMEM((CK_E, 16), _f32),
        pltpu.VMEM((CK_E, 16), _f32),
        pltpu.VMEM((16,), _f32),
        pltpu.VMEM_SHARED((N, 16), _f32),
        pltpu.SemaphoreType.DMA,
        pltpu.SemaphoreType.DMA,
        pltpu.SemaphoreType.DMA,
        pltpu.SemaphoreType.DMA,
    ],
)
def _e_kernel(src3_hbm, dst3_hbm, el_hbm, er_hbm, m_hbm, z16_hbm,
              w_hbm, den0_hbm, den1_hbm,
              idx_s2d, idx_d2d, ab0, ab1, bb0, bb1, wb0, wb1, mvec, den_sh,
              sem_a, sem_b, sem_w, semsc):
    c = lax.axis_index("c")
    s = lax.axis_index("s")
    wid = s * NC + c

    pltpu.sync_copy(z16_hbm.at[pl.ds(s * RPS, RPS)], den_sh.at[pl.ds(s * RPS, RPS)])

    @pl.when(s == NS - 1)
    def _():
        pltpu.sync_copy(z16_hbm.at[pl.ds(NS * RPS, TAIL)],
                        den_sh.at[pl.ds(NS * RPS, TAIL)])

    pltpu.sync_copy(m_hbm, mvec)
    pltpu.sync_copy(src3_hbm.at[wid], idx_s2d)
    pltpu.sync_copy(dst3_hbm.at[wid], idx_d2d)
    plsc.subcore_barrier()

    mv = mvec[...]

    def issue_gathers(j, abx, bbx):
        pltpu.async_copy(el_hbm.at[idx_s2d.at[j]], abx, sem_a)
        pltpu.async_copy(er_hbm.at[idx_d2d.at[j]], bbx, sem_b)

    def drain_gathers(abx, bbx):
        pltpu.make_async_copy(el_hbm.at[idx_s2d.at[0]], abx, sem_a).wait()
        pltpu.make_async_copy(er_hbm.at[idx_d2d.at[0]], bbx, sem_b).wait()

    def drain_wsc(wbx):
        pltpu.make_async_copy(wbx, w_hbm.at[pl.ds(0, CK_E)], sem_w).wait()
        pltpu.make_async_copy(wbx, den_sh.at[idx_d2d.at[0]], semsc).wait()

    def compute_and_store(j, abx, bbx, wbx):
        @plsc.parallel_loop(0, CK_E, unroll=8)
        def vbody(i):
            a = abx[i, :]
            b = bbx[i, :]
            e = a + b
            e = jnp.maximum(e, 0.2 * e)
            cb = mv + b
            cb = jnp.maximum(cb, 0.2 * cb)
            wbx[i, :] = jnp.exp(e - cb)
        pltpu.async_copy(wbx, w_hbm.at[pl.ds(wid * EPW + j * CK_E, CK_E)], sem_w)
        pltpu.async_copy(wbx, den_sh.at[idx_d2d.at[j]], semsc, add=True)

    issue_gathers(0, ab0, bb0)

    def pair(t, carry):
        j0 = 2 * t
        drain_gathers(ab0, bb0)

        @pl.when(t > 0)
        def _():
            drain_wsc(wb1)

        issue_gathers(j0 + 1, ab1, bb1)
        compute_and_store(j0, ab0, bb0, wb0)

        drain_gathers(ab1, bb1)
        drain_wsc(wb0)

        @pl.when(t < NIT_E // 2 - 1)
        def _():
            issue_gathers(j0 + 2, ab0, bb0)

        compute_and_store(j0 + 1, ab1, bb1, wb1)
        return carry

    lax.fori_loop(0, NIT_E // 2, pair, 0)
    drain_wsc(wb1)

    plsc.subcore_barrier()

    @pl.when(c == 0)
    def _():
        pltpu.sync_copy(den_sh.at[pl.ds(s * RPS, RPS)], den0_hbm.at[pl.ds(s * RPS, RPS)])

        @pl.when(s == NS - 1)
        def _():
            pltpu.sync_copy(den_sh.at[pl.ds(NS * RPS, TAIL)],
                            den0_hbm.at[pl.ds(NS * RPS, TAIL)])

    @pl.when(c == 1)
    def _():
        pltpu.sync_copy(den_sh.at[pl.ds(s * RPS, RPS)], den1_hbm.at[pl.ds(s * RPS, RPS)])

        @pl.when(s == NS - 1)
        def _():
            pltpu.sync_copy(den_sh.at[pl.ds(NS * RPS, TAIL)],
                            den1_hbm.at[pl.ds(NS * RPS, TAIL)])


@functools.partial(
    pl.kernel,
    out_type=[
        jax.ShapeDtypeStruct((N, H_DIM), _f32),
        jax.ShapeDtypeStruct((N, H_DIM), _f32),
    ],
    mesh=_MESH,
    compiler_params=pltpu.CompilerParams(use_tc_tiling_on_sc=False),
    scratch_types=[
        pltpu.VMEM((NIT_M, CK_M), jnp.int32),
        pltpu.VMEM((NIT_M, CK_M), jnp.int32),
        pltpu.VMEM((CK_M, H_DIM), _f32),
        pltpu.VMEM((CK_M, H_DIM), _f32),
        pltpu.VMEM((CK_M, 16), _f32),
        pltpu.VMEM((CK_M, 16), _f32),
        pltpu.VMEM((CK_M, 16), _f32),
        pltpu.VMEM((CK_M, 16), _f32),
        pltpu.VMEM_SHARED((N, H_DIM), _f32),
        pltpu.SemaphoreType.DMA,
        pltpu.SemaphoreType.DMA,
        pltpu.SemaphoreType.DMA,
        pltpu.SemaphoreType.DMA,
    ],
)
def _m_kernel(src3_hbm, dst3_hbm, wh_hbm, w_hbm, r_hbm, z128_hbm,
              o0_hbm, o1_hbm,
              idx_s2d, idx_d2d, gb0, gb1, wb0, wb1, rg0, rg1, out_sh,
              sem_wh, sem_r, sem_w, semsc):
    c = lax.axis_index("c")
    s = lax.axis_index("s")
    wid = s * NC + c

    pltpu.sync_copy(z128_hbm.at[pl.ds(s * RPS, RPS)], out_sh.at[pl.ds(s * RPS, RPS)])

    @pl.when(s == NS - 1)
    def _():
        pltpu.sync_copy(z128_hbm.at[pl.ds(NS * RPS, TAIL)],
                        out_sh.at[pl.ds(NS * RPS, TAIL)])

    pltpu.sync_copy(src3_hbm.at[wid], idx_s2d)
    pltpu.sync_copy(dst3_hbm.at[wid], idx_d2d)
    plsc.subcore_barrier()

    def issue_gathers(j, gb, wbx, rgx):
        pltpu.async_copy(wh_hbm.at[idx_s2d.at[j]], gb, sem_wh)
        pltpu.async_copy(r_hbm.at[idx_d2d.at[j]], rgx, sem_r)
        pltpu.async_copy(w_hbm.at[pl.ds(wid * EPW + j * CK_M, CK_M)], wbx, sem_w)

    def drain_gathers(gb, wbx, rgx):
        pltpu.make_async_copy(wh_hbm.at[idx_s2d.at[0]], gb, sem_wh).wait()
        pltpu.make_async_copy(r_hbm.at[idx_d2d.at[0]], rgx, sem_r).wait()
        pltpu.make_async_copy(w_hbm.at[pl.ds(0, CK_M)], wbx, sem_w).wait()

    def drain_scatter():
        pltpu.make_async_copy(gb0, out_sh.at[idx_d2d.at[0]], semsc).wait()

    def compute_and_scatter(j, gb, wbx, rgx):
        @plsc.parallel_loop(0, CK_M, unroll=4)
        def vbody(i):
            aw = wbx[i, :] * rgx[i, :]
            for hd in range(HEADS):
                gb[i, 16 * hd:16 * (hd + 1)] = (
                    gb[i, 16 * hd:16 * (hd + 1)] * aw[hd])

        pltpu.async_copy(gb, out_sh.at[idx_d2d.at[j]], semsc, add=True)

    issue_gathers(0, gb0, wb0, rg0)

    def pair(t, carry):
        j0 = 2 * t
        drain_gathers(gb0, wb0, rg0)

        @pl.when(t > 0)
        def _():
            drain_scatter()

        issue_gathers(j0 + 1, gb1, wb1, rg1)
        compute_and_scatter(j0, gb0, wb0, rg0)

        drain_gathers(gb1, wb1, rg1)
        drain_scatter()
        issue_gathers(j0 + 2, gb0, wb0, rg0)
        compute_and_scatter(j0 + 1, gb1, wb1, rg1)
        return carry

    lax.fori_loop(0, NIT_M // 2, pair, 0)
    drain_gathers(gb0, wb0, rg0)
    drain_scatter()
    compute_and_scatter(NIT_M - 1, gb0, wb0, rg0)
    drain_scatter()
    plsc.subcore_barrier()

    @pl.when(c == 0)
    def _():
        pltpu.sync_copy(out_sh.at[pl.ds(s * RPS, RPS)], o0_hbm.at[pl.ds(s * RPS, RPS)])

        @pl.when(s == NS - 1)
        def _():
            pltpu.sync_copy(out_sh.at[pl.ds(NS * RPS, TAIL)],
                            o0_hbm.at[pl.ds(NS * RPS, TAIL)])

    @pl.when(c == 1)
    def _():
        pltpu.sync_copy(out_sh.at[pl.ds(s * RPS, RPS)], o1_hbm.at[pl.ds(s * RPS, RPS)])

        @pl.when(s == NS - 1)
        def _():
            pltpu.sync_copy(out_sh.at[pl.ds(NS * RPS, TAIL)],
                            o1_hbm.at[pl.ds(NS * RPS, TAIL)])


def kernel(h, edge_index, node_graph_id, snorm_n, W_embed, Ws, a_srcs, a_dsts,
           W_ro, W_pred, b_pred):
    src = edge_index[0]
    dst = edge_index[1]

    head_of = jnp.arange(H_DIM, dtype=jnp.int32) // DH
    delta = (head_of[:, None] == jnp.arange(HEADS)[None, :]).astype(_f32)
    asv = a_srcs.reshape(L, H_DIM)[:, :, None] * delta[None]
    adv = a_dsts.reshape(L, H_DIM)[:, :, None] * delta[None]
    aser = jnp.concatenate([asv, asv, adv, adv], axis=2)

    zeros16 = jnp.zeros((N, 16), _f32)
    zeros128 = jnp.zeros((N, H_DIM), _f32)
    snorm = snorm_n.astype(_f32)
    src3 = src.reshape(NW, NIT_M, CK_M)
    dst3 = dst.reshape(NW, NIT_M, CK_M)
    src3e = src.reshape(NW, NIT_E, CK_E)
    dst3e = dst.reshape(NW, NIT_E, CK_E)

    x0 = _x0_call(h.astype(_f32), W_embed)

    def layer(l, carry):
        x, o0, o1 = carry
        W_l = lax.dynamic_index_in_dim(Ws, l, 0, keepdims=False)
        aser_l = lax.dynamic_index_in_dim(aser, l, 0, keepdims=False)
        x, wh, el, er, m = _al_call(x, o0, o1, snorm, W_l, aser_l)
        w, den0, den1 = _e_kernel(src3e, dst3e, el, er, m.reshape(16), zeros16)
        r = _r_call(den0, den1)
        o0, o1 = _m_kernel(src3, dst3, wh, w, r, zeros128)
        return (x, o0, o1)

    x, o0, o1 = lax.fori_loop(0, L, layer, (x0, zeros128, zeros128))

    gid2d = node_graph_id.reshape(N, 1)
    b2d = b_pred.reshape(1, 1)
    return _ro_call(x, o0, o1, snorm, gid2d, W_ro, W_pred, b2d)
